# Initial kernel scaffold; baseline (speedup 1.0000x reference)
import jax
import jax.numpy as jnp
from jax.experimental import pallas as pl


def kernel(x, edge_index, e_id, attr, W_gcn, b_gcn, W_l, b_l, W_r):
    raise NotImplementedError("write your pallas kernel here")


# speedup vs baseline: 9.0630x; 9.0630x over previous
import functools

import jax
import jax.numpy as jnp
from jax import lax
from jax.experimental import pallas as pl
from jax.experimental.pallas import tpu as pltpu
from jax.experimental.pallas import tpu_sc as plsc

N_T = 10000
D = 128
E = 320000
NC = 2
NS = 16
NW = NC * NS
CK = 128
CH = 80
EPW = CH * CK
E_PAD = NW * EPW
N_ACC = 10240
TRASH = N_T
SLC = N_ACC // NS

_mesh = plsc.VectorSubcoreMesh(core_axis_name="c", subcore_axis_name="s")
_f32 = jnp.float32


def _ids():
    cid = lax.axis_index("c")
    sid = lax.axis_index("s")
    return cid, sid, sid * NC + cid


@functools.partial(
    pl.kernel,
    out_type=[
        jax.ShapeDtypeStruct((NW, CH, CK), _f32),
        jax.ShapeDtypeStruct((NC, N_ACC), _f32),
        jax.ShapeDtypeStruct((NC, N_ACC), _f32),
    ],
    mesh=_mesh,
    scratch_types=[
        pltpu.VMEM((CH, CK), jnp.int32),
        pltpu.VMEM((CH, CK), jnp.int32),
        pltpu.VMEM((CH, CK), _f32),
        pltpu.VMEM((CK,), _f32),
        pltpu.VMEM((SLC,), _f32),
        pltpu.VMEM_SHARED((N_ACC,), _f32),
        pltpu.VMEM_SHARED((N_ACC,), _f32),
        pltpu.SemaphoreType.DMA,
        pltpu.SemaphoreType.DMA,
    ],
)
def _pass_a(attr_hbm, eid3_hbm, col3_hbm, w3_out, degp_out, cntp_out,
            eid_v, col_v, w_v, ones_v, zb, deg_sh, cnt_sh, sem_g, sem_s):
    cid, sid, wid = _ids()
    base = sid * SLC

    def _zb(i, _):
        zb[pl.ds(16 * i, 16)] = jnp.zeros((16,), _f32)
        return _
    lax.fori_loop(0, SLC // 16, _zb, None)
    for k in range(CK // 16):
        ones_v[pl.ds(16 * k, 16)] = jnp.ones((16,), _f32)
    pltpu.sync_copy(zb, deg_sh.at[pl.ds(base, SLC)])
    pltpu.sync_copy(zb, cnt_sh.at[pl.ds(base, SLC)])
    pltpu.sync_copy(eid3_hbm.at[wid], eid_v)
    pltpu.sync_copy(col3_hbm.at[wid], col_v)
    plsc.subcore_barrier()

    GP = 8
    for g in range(CH // GP):
        gds = []
        for b in range(GP):
            j = GP * g + b
            gds.append(pltpu.async_copy(attr_hbm.at[eid_v.at[j]], w_v.at[j], sem_g))
        for d in gds:
            d.wait()
        sds = []
        for b in range(GP):
            j = GP * g + b
            sds.append(pltpu.async_copy(w_v.at[j], deg_sh.at[col_v.at[j]], sem_s, add=True))
            sds.append(pltpu.async_copy(ones_v, cnt_sh.at[col_v.at[j]], sem_s, add=True))
        for d in sds:
            d.wait()

    pltpu.sync_copy(w_v, w3_out.at[wid])
    plsc.subcore_barrier()
    pltpu.sync_copy(deg_sh.at[pl.ds(base, SLC)], degp_out.at[cid, pl.ds(base, SLC)])
    pltpu.sync_copy(cnt_sh.at[pl.ds(base, SLC)], cntp_out.at[cid, pl.ds(base, SLC)])


def _zero_acc(zb2, acc_sh, sid, sem):
    def _zb(i, _):
        for k in range(D // 16):
            zb2[i, pl.ds(16 * k, 16)] = jnp.zeros((16,), _f32)
        return _
    lax.fori_loop(0, 40, _zb, None)
    base = sid * SLC
    ds = []
    for t in range(SLC // 40):
        ds.append(pltpu.async_copy(zb2, acc_sh.at[pl.ds(base + t * 40, 40), :], sem))
    for d in ds:
        d.wait()


@functools.partial(
    pl.kernel,
    out_type=[jax.ShapeDtypeStruct((NC, N_ACC, D), _f32)],
    mesh=_mesh,
    scratch_types=[
        pltpu.VMEM((2, 2, CK), jnp.int32),
        pltpu.VMEM((2, 2, CK), jnp.int32),
        pltpu.VMEM((2, 2, CK), _f32),
        pltpu.VMEM((CK, D), _f32),
        pltpu.VMEM((CK, D), _f32),
        pltpu.VMEM((40, D), _f32),
        pltpu.VMEM_SHARED((N_ACC, D), _f32),
        pltpu.SemaphoreType.DMA,
        pltpu.SemaphoreType.DMA,
        pltpu.SemaphoreType.DMA,
    ],
)
def _pass_b(h2_hbm, row3_hbm, col3_hbm, w3_hbm, acc_out,
            row4, col4, w4, bufA, bufB, zb2, acc_sh, semI, semGA, semGB):
    cid, sid, wid = _ids()
    base = sid * SLC
    _zero_acc(zb2, acc_sh, sid, semI)
    plsc.subcore_barrier()

    def _idx_descs(p, j0):
        return [
            pltpu.make_async_copy(row3_hbm.at[wid, j0], row4.at[p, 0], semI),
            pltpu.make_async_copy(col3_hbm.at[wid, j0], col4.at[p, 0], semI),
            pltpu.make_async_copy(w3_hbm.at[wid, j0], w4.at[p, 0], semI),
            pltpu.make_async_copy(row3_hbm.at[wid, j0 + 1], row4.at[p, 1], semI),
            pltpu.make_async_copy(col3_hbm.at[wid, j0 + 1], col4.at[p, 1], semI),
            pltpu.make_async_copy(w3_hbm.at[wid, j0 + 1], w4.at[p, 1], semI),
        ]

    def _scale(buf, p, h):
        def _grp(k, _):
            wvec = w4[p, h, pl.ds(16 * k, 16)]
            for l in range(16):
                s = wvec[l]
                e = 16 * k + l
                for q in range(D // 16):
                    buf[e, pl.ds(16 * q, 16)] = buf[e, pl.ds(16 * q, 16)] * s
            return _
        lax.fori_loop(0, CK // 16, _grp, None)

    for d in _idx_descs(0, 0):
        d.start()

    def _body(jj, _):
        p = jnp.bitwise_and(jj, 1)
        q = 1 - p
        j0 = 2 * jj
        for d in _idx_descs(p, j0):
            d.wait()
        pltpu.async_copy(h2_hbm.at[row4.at[p, 0]], bufA, semGA)
        pltpu.async_copy(h2_hbm.at[row4.at[p, 1]], bufB, semGB)

        @pl.when(jj < CH // 2 - 1)
        def _():
            for d in _idx_descs(q, j0 + 2):
                d.start()

        pltpu.make_async_copy(h2_hbm.at[row4.at[p, 0]], bufA, semGA).wait()
        _scale(bufA, p, 0)
        pltpu.sync_copy(bufA, acc_sh.at[col4.at[p, 0]], add=True)
        pltpu.make_async_copy(h2_hbm.at[row4.at[p, 1]], bufB, semGB).wait()
        _scale(bufB, p, 1)
        pltpu.sync_copy(bufB, acc_sh.at[col4.at[p, 1]], add=True)
        return _

    lax.fori_loop(0, CH // 2, _body, None)
    plsc.subcore_barrier()
    pltpu.sync_copy(acc_sh.at[pl.ds(base, SLC), :], acc_out.at[cid, pl.ds(base, SLC), :])


@functools.partial(
    pl.kernel,
    out_type=[jax.ShapeDtypeStruct((NC, N_ACC, D), _f32)],
    mesh=_mesh,
    scratch_types=[
        pltpu.VMEM((2, 2, CK), jnp.int32),
        pltpu.VMEM((2, 2, CK), jnp.int32),
        pltpu.VMEM((CK, D), _f32),
        pltpu.VMEM((CK, D), _f32),
        pltpu.VMEM((40, D), _f32),
        pltpu.VMEM_SHARED((N_ACC, D), _f32),
        pltpu.SemaphoreType.DMA,
        pltpu.SemaphoreType.DMA,
        pltpu.SemaphoreType.DMA,
    ],
)
def _pass_c(gcn_hbm, row3_hbm, col3_hbm, sum_out,
            row4, col4, bufA, bufB, zb2, acc_sh, semI, semGA, semGB):
    cid, sid, wid = _ids()
    base = sid * SLC
    _zero_acc(zb2, acc_sh, sid, semI)
    plsc.subcore_barrier()

    def _idx_descs(p, j0):
        return [
            pltpu.make_async_copy(row3_hbm.at[wid, j0], row4.at[p, 0], semI),
            pltpu.make_async_copy(col3_hbm.at[wid, j0], col4.at[p, 0], semI),
            pltpu.make_async_copy(row3_hbm.at[wid, j0 + 1], row4.at[p, 1], semI),
            pltpu.make_async_copy(col3_hbm.at[wid, j0 + 1], col4.at[p, 1], semI),
        ]

    for d in _idx_descs(0, 0):
        d.start()

    def _body(jj, _):
        p = jnp.bitwise_and(jj, 1)
        q = 1 - p
        j0 = 2 * jj
        for d in _idx_descs(p, j0):
            d.wait()
        pltpu.async_copy(gcn_hbm.at[row4.at[p, 0]], bufA, semGA)
        pltpu.async_copy(gcn_hbm.at[row4.at[p, 1]], bufB, semGB)

        @pl.when(jj < CH // 2 - 1)
        def _():
            for d in _idx_descs(q, j0 + 2):
                d.start()

        pltpu.make_async_copy(gcn_hbm.at[row4.at[p, 0]], bufA, semGA).wait()
        pltpu.sync_copy(bufA, acc_sh.at[col4.at[p, 0]], add=True)
        pltpu.make_async_copy(gcn_hbm.at[row4.at[p, 1]], bufB, semGB).wait()
        pltpu.sync_copy(bufB, acc_sh.at[col4.at[p, 1]], add=True)
        return _

    lax.fori_loop(0, CH // 2, _body, None)
    plsc.subcore_barrier()
    pltpu.sync_copy(acc_sh.at[pl.ds(base, SLC), :], sum_out.at[cid, pl.ds(base, SLC), :])


_RB = 1000


def _tc12_body(x_ref, w_ref, degA_ref, degB_ref, h2_ref, hd_ref, dinv_ref):
    h = jnp.dot(x_ref[...], w_ref[...], preferred_element_type=_f32)
    deg = 1.0 + degA_ref[...] + degB_ref[...]
    dinv = lax.rsqrt(deg)
    h2_ref[...] = h * dinv
    hd_ref[...] = h / deg
    dinv_ref[...] = dinv


def _tc12(x_t, W_gcn, degA, degB):
    return pl.pallas_call(
        _tc12_body,
        grid=(N_T // _RB,),
        in_specs=[
            pl.BlockSpec((_RB, D), lambda i: (i, 0)),
            pl.BlockSpec((D, D), lambda i: (0, 0)),
            pl.BlockSpec((_RB, 1), lambda i: (i, 0)),
            pl.BlockSpec((_RB, 1), lambda i: (i, 0)),
        ],
        out_specs=[
            pl.BlockSpec((_RB, D), lambda i: (i, 0)),
            pl.BlockSpec((_RB, D), lambda i: (i, 0)),
            pl.BlockSpec((_RB, 1), lambda i: (i, 0)),
        ],
        out_shape=[
            jax.ShapeDtypeStruct((N_T, D), _f32),
            jax.ShapeDtypeStruct((N_T, D), _f32),
            jax.ShapeDtypeStruct((N_T, 1), _f32),
        ],
    )(x_t, W_gcn, degA, degB)


def _tc3_body(dinv_ref, accA_ref, accB_ref, hd_ref, bg_ref, gcn_ref):
    gcn_ref[...] = (dinv_ref[...] * (accA_ref[...] + accB_ref[...])
                    + hd_ref[...] + bg_ref[...])


def _tc3(dinv, accA, accB, hd, bg):
    return pl.pallas_call(
        _tc3_body,
        grid=(N_T // _RB,),
        in_specs=[
            pl.BlockSpec((_RB, 1), lambda i: (i, 0)),
            pl.BlockSpec((_RB, D), lambda i: (i, 0)),
            pl.BlockSpec((_RB, D), lambda i: (i, 0)),
            pl.BlockSpec((_RB, D), lambda i: (i, 0)),
            pl.BlockSpec((1, D), lambda i: (0, 0)),
        ],
        out_specs=pl.BlockSpec((_RB, D), lambda i: (i, 0)),
        out_shape=jax.ShapeDtypeStruct((N_T, D), _f32),
    )(dinv, accA, accB, hd, bg)


def _tc4_body(sA_ref, sB_ref, cntA_ref, cntB_ref, gcn_ref, wl_ref, wr_ref,
              bl_ref, out_ref):
    cnt = jnp.maximum(cntA_ref[...] + cntB_ref[...], 1.0)
    mean = (sA_ref[...] + sB_ref[...]) / cnt
    o = (jnp.dot(mean, wl_ref[...], preferred_element_type=_f32)
         + jnp.dot(gcn_ref[...], wr_ref[...], preferred_element_type=_f32)
         + bl_ref[...])
    ss = jnp.sum(o * o, axis=-1, keepdims=True)
    nrm = jnp.sqrt(jnp.maximum(ss, 1e-24))
    out_ref[...] = o / jnp.maximum(nrm, 1e-12)


def _tc4(sA, sB, cntA, cntB, gcn, WlT, WrT, bl):
    return pl.pallas_call(
        _tc4_body,
        grid=(N_T // _RB,),
        in_specs=[
            pl.BlockSpec((_RB, D), lambda i: (i, 0)),
            pl.BlockSpec((_RB, D), lambda i: (i, 0)),
            pl.BlockSpec((_RB, 1), lambda i: (i, 0)),
            pl.BlockSpec((_RB, 1), lambda i: (i, 0)),
            pl.BlockSpec((_RB, D), lambda i: (i, 0)),
            pl.BlockSpec((D, D), lambda i: (0, 0)),
            pl.BlockSpec((D, D), lambda i: (0, 0)),
            pl.BlockSpec((1, D), lambda i: (0, 0)),
        ],
        out_specs=pl.BlockSpec((_RB, D), lambda i: (i, 0)),
        out_shape=jax.ShapeDtypeStruct((N_T, D), _f32),
    )(sA, sB, cntA, cntB, gcn, WlT, WrT, bl)


def kernel(x, edge_index, e_id, attr, W_gcn, b_gcn, W_l, b_l, W_r):
    row = edge_index[0]
    col = edge_index[1]
    pad = E_PAD - E
    row3 = jnp.concatenate([row, jnp.zeros((pad,), jnp.int32)]).reshape(NW, CH, CK)
    col3 = jnp.concatenate([col, jnp.full((pad,), TRASH, jnp.int32)]).reshape(NW, CH, CK)
    eid3 = jnp.concatenate([e_id, jnp.zeros((pad,), jnp.int32)]).reshape(NW, CH, CK)
    x_t = x[:N_T]

    w3, degp, cntp = _pass_a(attr, eid3, col3)
    degA = degp[0, :N_T, None]
    degB = degp[1, :N_T, None]
    h2, hd, dinv = _tc12(x_t, W_gcn, degA, degB)
    (accp,) = _pass_b(h2, row3, col3, w3)
    gcn = _tc3(dinv, accp[0, :N_T], accp[1, :N_T], hd, b_gcn[None, :])
    (sump,) = _pass_c(gcn, row3, col3)
    out = _tc4(sump[0, :N_T], sump[1, :N_T], cntp[0, :N_T, None],
               cntp[1, :N_T, None], gcn, W_l.T, W_r.T, b_l[None, :])
    return out


# speedup vs baseline: 17.2729x; 1.9059x over previous
import functools

import jax
import jax.numpy as jnp
from jax import lax
from jax.experimental import pallas as pl
from jax.experimental.pallas import tpu as pltpu
from jax.experimental.pallas import tpu_sc as plsc

N_T = 10000
D = 128
E = 320000
NC = 2
NS = 16
NW = NC * NS
CK = 80
CH = 125
N_ACC = 10240
SLC = N_ACC // NS

_mesh = plsc.VectorSubcoreMesh(core_axis_name="c", subcore_axis_name="s")
_f32 = jnp.float32


def _ids():
    cid = lax.axis_index("c")
    sid = lax.axis_index("s")
    return cid, sid, sid * NC + cid


@functools.partial(
    pl.kernel,
    out_type=[
        jax.ShapeDtypeStruct((NW, CH, CK), _f32),
        jax.ShapeDtypeStruct((NC, N_ACC), _f32),
        jax.ShapeDtypeStruct((NC, N_ACC), _f32),
    ],
    mesh=_mesh,
    scratch_types=[
        pltpu.VMEM((8, CK), jnp.int32),
        pltpu.VMEM((8, CK), jnp.int32),
        pltpu.VMEM((CH, CK), _f32),
        pltpu.VMEM((CK,), _f32),
        pltpu.VMEM((SLC,), _f32),
        pltpu.VMEM_SHARED((N_ACC,), _f32),
        pltpu.VMEM_SHARED((N_ACC,), _f32),
        pltpu.SemaphoreType.DMA,
        pltpu.SemaphoreType.DMA((2,)),
        pltpu.SemaphoreType.DMA((2,)),
    ],
)
def _pass_a(attr_hbm, eid3_hbm, col3_hbm, w3_out, degp_out, cntp_out,
            eid8, col8, w_v, ones_v, zb, deg_sh, cnt_sh,
            semI, semG, semS):
    cid, sid, wid = _ids()
    base = sid * SLC

    def _zb(i, _):
        zb[pl.ds(16 * i, 16)] = jnp.zeros((16,), _f32)
        return _
    lax.fori_loop(0, SLC // 16, _zb, None)
    for k in range(CK // 16):
        ones_v[pl.ds(16 * k, 16)] = jnp.ones((16,), _f32)
    pltpu.sync_copy(zb, deg_sh.at[pl.ds(base, SLC)])
    pltpu.sync_copy(zb, cnt_sh.at[pl.ds(base, SLC)])
    plsc.subcore_barrier()

    def _idx(p, j):
        return [
            pltpu.make_async_copy(eid3_hbm.at[wid, j], eid8.at[p], semI),
            pltpu.make_async_copy(col3_hbm.at[wid, j], col8.at[p], semI),
        ]

    def _gather(j, sg):
        return pltpu.make_async_copy(attr_hbm.at[eid8.at[jnp.bitwise_and(j, 7)]],
                                     w_v.at[j], sg)

    def _scats(j, ss):
        s = jnp.bitwise_and(j, 7)
        return [
            pltpu.make_async_copy(w_v.at[j], deg_sh.at[col8.at[s]], ss),
            pltpu.make_async_copy(ones_v, cnt_sh.at[col8.at[s]], ss),
        ]

    for j in range(6):
        for d in _idx(j, j):
            d.start()
    for d in _idx(0, 0):
        d.wait()
    for d in _idx(1, 1):
        d.wait()
    _gather(0, semG.at[0]).start()
    _gather(1, semG.at[1]).start()

    def _body(j, _):
        sg = jnp.bitwise_and(j, 1)

        @pl.when(j >= 2)
        def _():
            for d in _scats(j - 2, semS.at[sg]):
                d.wait()

        @pl.when(j + 6 < CH)
        def _():
            for d in _idx(jnp.bitwise_and(j + 6, 7), j + 6):
                d.start()

        _gather(j, semG.at[sg]).wait()
        for d in _scats(j, semS.at[sg]):
            d.start(add=True)

        @pl.when(j + 2 < CH)
        def _():
            for d in _idx(jnp.bitwise_and(j + 2, 7), j + 2):
                d.wait()
            _gather(j + 2, semG.at[sg]).start()
        return _

    lax.fori_loop(0, CH, _body, None)
    for d in _scats(CH - 2, semS.at[1]):
        d.wait()
    for d in _scats(CH - 1, semS.at[0]):
        d.wait()
    pltpu.sync_copy(w_v, w3_out.at[wid])
    plsc.subcore_barrier()
    pltpu.sync_copy(deg_sh.at[pl.ds(base, SLC)], degp_out.at[cid, pl.ds(base, SLC)])
    pltpu.sync_copy(cnt_sh.at[pl.ds(base, SLC)], cntp_out.at[cid, pl.ds(base, SLC)])


def _zero_acc(zb2, acc_sh, sid, sem):
    zr = zb2.shape[0]

    def _zb(i, _):
        for k in range(D // 16):
            zb2[i, pl.ds(16 * k, 16)] = jnp.zeros((16,), _f32)
        return _
    lax.fori_loop(0, zr, _zb, None)
    base = sid * SLC
    ds = []
    for t in range(SLC // zr):
        ds.append(pltpu.async_copy(zb2, acc_sh.at[pl.ds(base + t * zr, zr), :], sem))
    for d in ds:
        d.wait()


def _agg_pass(table_hbm, row3_hbm, col3_hbm, w3_hbm, out_ref,
              row8, col8, w8, buf4, zb2, acc_sh, semI, semG, semS):
    cid, sid, wid = _ids()
    base = sid * SLC
    _zero_acc(zb2, acc_sh, sid, semI)
    plsc.subcore_barrier()

    def _idx(p, j):
        ds = [
            pltpu.make_async_copy(row3_hbm.at[wid, j], row8.at[p], semI),
            pltpu.make_async_copy(col3_hbm.at[wid, j], col8.at[p], semI),
        ]
        if w8 is not None:
            ds.append(pltpu.make_async_copy(w3_hbm.at[wid, j], w8.at[p], semI))
        return ds

    def _gather(j, sg):
        return pltpu.make_async_copy(
            table_hbm.at[row8.at[jnp.bitwise_and(j, 7)]],
            buf4.at[jnp.bitwise_and(j, 3)], sg)

    def _scat(j, ss):
        return pltpu.make_async_copy(
            buf4.at[jnp.bitwise_and(j, 3)],
            acc_sh.at[col8.at[jnp.bitwise_and(j, 7)]], ss)

    def _scale(j):
        b = jnp.bitwise_and(j, 3)
        s = jnp.bitwise_and(j, 7)

        def _grp(k, _):
            wvec = w8[s, pl.ds(16 * k, 16)]
            for l in range(16):
                sc = wvec[l]
                e = 16 * k + l
                for q in range(D // 16):
                    buf4[b, e, pl.ds(16 * q, 16)] = buf4[b, e, pl.ds(16 * q, 16)] * sc
            return _
        lax.fori_loop(0, CK // 16, _grp, None)

    for j in range(6):
        for d in _idx(j, j):
            d.start()
    for d in _idx(0, 0):
        d.wait()
    for d in _idx(1, 1):
        d.wait()
    _gather(0, semG.at[0]).start()
    _gather(1, semG.at[1]).start()

    def _body(j, _):
        sg = jnp.bitwise_and(j, 1)

        @pl.when(j >= 2)
        def _():
            _scat(j - 2, semS.at[sg]).wait()

        @pl.when(j + 6 < CH)
        def _():
            for d in _idx(jnp.bitwise_and(j + 6, 7), j + 6):
                d.start()

        _gather(j, semG.at[sg]).wait()
        if w8 is not None:
            _scale(j)
        _scat(j, semS.at[sg]).start(add=True)

        @pl.when(j + 2 < CH)
        def _():
            for d in _idx(jnp.bitwise_and(j + 2, 7), j + 2):
                d.wait()
            _gather(j + 2, semG.at[sg]).start()
        return _

    lax.fori_loop(0, CH, _body, None)
    _scat(CH - 2, semS.at[1]).wait()
    _scat(CH - 1, semS.at[0]).wait()
    plsc.subcore_barrier()
    pltpu.sync_copy(acc_sh.at[pl.ds(base, SLC), :], out_ref.at[cid, pl.ds(base, SLC), :])


@functools.partial(
    pl.kernel,
    out_type=[jax.ShapeDtypeStruct((NC, N_ACC, D), _f32)],
    mesh=_mesh,
    scratch_types=[
        pltpu.VMEM((8, CK), jnp.int32),
        pltpu.VMEM((8, CK), jnp.int32),
        pltpu.VMEM((8, CK), _f32),
        pltpu.VMEM((4, CK, D), _f32),
        pltpu.VMEM((20, D), _f32),
        pltpu.VMEM_SHARED((N_ACC, D), _f32),
        pltpu.SemaphoreType.DMA,
        pltpu.SemaphoreType.DMA((2,)),
        pltpu.SemaphoreType.DMA((2,)),
    ],
)
def _pass_b(h2_hbm, row3_hbm, col3_hbm, w3_hbm, acc_out,
            row8, col8, w8, buf4, zb2, acc_sh, semI, semG, semS):
    _agg_pass(h2_hbm, row3_hbm, col3_hbm, w3_hbm, acc_out,
              row8, col8, w8, buf4, zb2, acc_sh, semI, semG, semS)


@functools.partial(
    pl.kernel,
    out_type=[jax.ShapeDtypeStruct((NC, N_ACC, D), _f32)],
    mesh=_mesh,
    scratch_types=[
        pltpu.VMEM((8, CK), jnp.int32),
        pltpu.VMEM((8, CK), jnp.int32),
        pltpu.VMEM((4, CK, D), _f32),
        pltpu.VMEM((20, D), _f32),
        pltpu.VMEM_SHARED((N_ACC, D), _f32),
        pltpu.SemaphoreType.DMA,
        pltpu.SemaphoreType.DMA((2,)),
        pltpu.SemaphoreType.DMA((2,)),
    ],
)
def _pass_c(gcn_hbm, row3_hbm, col3_hbm, sum_out,
            row8, col8, buf4, zb2, acc_sh, semI, semG, semS):
    _agg_pass(gcn_hbm, row3_hbm, col3_hbm, None, sum_out,
              row8, col8, None, buf4, zb2, acc_sh, semI, semG, semS)


_RB = 1000


def _tc12_body(x_ref, w_ref, degA_ref, degB_ref, h2_ref, hd_ref, dinv_ref):
    h = jnp.dot(x_ref[...], w_ref[...], preferred_element_type=_f32)
    deg = 1.0 + degA_ref[...] + degB_ref[...]
    dinv = lax.rsqrt(deg)
    h2_ref[...] = h * dinv
    hd_ref[...] = h / deg
    dinv_ref[...] = dinv


def _tc12(x_t, W_gcn, degA, degB):
    return pl.pallas_call(
        _tc12_body,
        grid=(N_T // _RB,),
        in_specs=[
            pl.BlockSpec((_RB, D), lambda i: (i, 0)),
            pl.BlockSpec((D, D), lambda i: (0, 0)),
            pl.BlockSpec((_RB, 1), lambda i: (i, 0)),
            pl.BlockSpec((_RB, 1), lambda i: (i, 0)),
        ],
        out_specs=[
            pl.BlockSpec((_RB, D), lambda i: (i, 0)),
            pl.BlockSpec((_RB, D), lambda i: (i, 0)),
            pl.BlockSpec((_RB, 1), lambda i: (i, 0)),
        ],
        out_shape=[
            jax.ShapeDtypeStruct((N_T, D), _f32),
            jax.ShapeDtypeStruct((N_T, D), _f32),
            jax.ShapeDtypeStruct((N_T, 1), _f32),
        ],
    )(x_t, W_gcn, degA, degB)


def _tc3_body(dinv_ref, accA_ref, accB_ref, hd_ref, bg_ref, gcn_ref):
    gcn_ref[...] = (dinv_ref[...] * (accA_ref[...] + accB_ref[...])
                    + hd_ref[...] + bg_ref[...])


def _tc3(dinv, accA, accB, hd, bg):
    return pl.pallas_call(
        _tc3_body,
        grid=(N_T // _RB,),
        in_specs=[
            pl.BlockSpec((_RB, 1), lambda i: (i, 0)),
            pl.BlockSpec((_RB, D), lambda i: (i, 0)),
            pl.BlockSpec((_RB, D), lambda i: (i, 0)),
            pl.BlockSpec((_RB, D), lambda i: (i, 0)),
            pl.BlockSpec((1, D), lambda i: (0, 0)),
        ],
        out_specs=pl.BlockSpec((_RB, D), lambda i: (i, 0)),
        out_shape=jax.ShapeDtypeStruct((N_T, D), _f32),
    )(dinv, accA, accB, hd, bg)


def _tc4_body(sA_ref, sB_ref, cntA_ref, cntB_ref, gcn_ref, wl_ref, wr_ref,
              bl_ref, out_ref):
    cnt = jnp.maximum(cntA_ref[...] + cntB_ref[...], 1.0)
    mean = (sA_ref[...] + sB_ref[...]) / cnt
    o = (jnp.dot(mean, wl_ref[...], preferred_element_type=_f32)
         + jnp.dot(gcn_ref[...], wr_ref[...], preferred_element_type=_f32)
         + bl_ref[...])
    ss = jnp.sum(o * o, axis=-1, keepdims=True)
    nrm = jnp.sqrt(jnp.maximum(ss, 1e-24))
    out_ref[...] = o / jnp.maximum(nrm, 1e-12)


def _tc4(sA, sB, cntA, cntB, gcn, WlT, WrT, bl):
    return pl.pallas_call(
        _tc4_body,
        grid=(N_T // _RB,),
        in_specs=[
            pl.BlockSpec((_RB, D), lambda i: (i, 0)),
            pl.BlockSpec((_RB, D), lambda i: (i, 0)),
            pl.BlockSpec((_RB, 1), lambda i: (i, 0)),
            pl.BlockSpec((_RB, 1), lambda i: (i, 0)),
            pl.BlockSpec((_RB, D), lambda i: (i, 0)),
            pl.BlockSpec((D, D), lambda i: (0, 0)),
            pl.BlockSpec((D, D), lambda i: (0, 0)),
            pl.BlockSpec((1, D), lambda i: (0, 0)),
        ],
        out_specs=pl.BlockSpec((_RB, D), lambda i: (i, 0)),
        out_shape=jax.ShapeDtypeStruct((N_T, D), _f32),
    )(sA, sB, cntA, cntB, gcn, WlT, WrT, bl)


def kernel(x, edge_index, e_id, attr, W_gcn, b_gcn, W_l, b_l, W_r):
    row3 = edge_index[0].reshape(NW, CH, CK)
    col3 = edge_index[1].reshape(NW, CH, CK)
    eid3 = e_id.reshape(NW, CH, CK)
    x_t = x[:N_T]

    w3, degp, cntp = _pass_a(attr, eid3, col3)
    degA = degp[0, :N_T, None]
    degB = degp[1, :N_T, None]
    h2, hd, dinv = _tc12(x_t, W_gcn, degA, degB)
    (accp,) = _pass_b(h2, row3, col3, w3)
    gcn = _tc3(dinv, accp[0, :N_T], accp[1, :N_T], hd, b_gcn[None, :])
    (sump,) = _pass_c(gcn, row3, col3)
    out = _tc4(sump[0, :N_T], sump[1, :N_T], cntp[0, :N_T, None],
               cntp[1, :N_T, None], gcn, W_l.T, W_r.T, b_l[None, :])
    return out


# speedup vs baseline: 29.0119x; 1.6796x over previous
import functools

import jax
import jax.numpy as jnp
from jax import lax
from jax.experimental import pallas as pl
from jax.experimental.pallas import tpu as pltpu
from jax.experimental.pallas import tpu_sc as plsc

N_T = 10000
D = 128
E = 320000
NC = 2
NS = 16
NW = NC * NS
CK = 80
CH = 125
N_ACC = 10240
SLC = N_ACC // NS

_mesh = plsc.VectorSubcoreMesh(core_axis_name="c", subcore_axis_name="s")
_f32 = jnp.float32


def _ids():
    cid = lax.axis_index("c")
    sid = lax.axis_index("s")
    return cid, sid, sid * NC + cid


@functools.partial(
    pl.kernel,
    out_type=[
        jax.ShapeDtypeStruct((NW, CH, CK), _f32),
        jax.ShapeDtypeStruct((NC, N_ACC), _f32),
        jax.ShapeDtypeStruct((NC, N_ACC), _f32),
    ],
    mesh=_mesh,
    scratch_types=[
        pltpu.VMEM((8, CK), jnp.int32),
        pltpu.VMEM((8, CK), jnp.int32),
        pltpu.VMEM((CH, CK), _f32),
        pltpu.VMEM((CK,), _f32),
        pltpu.VMEM((SLC,), _f32),
        pltpu.VMEM_SHARED((N_ACC,), _f32),
        pltpu.VMEM_SHARED((N_ACC,), _f32),
        pltpu.SemaphoreType.DMA,
        pltpu.SemaphoreType.DMA((2,)),
        pltpu.SemaphoreType.DMA((2,)),
    ],
)
def _pass_a(attr_hbm, eid3_hbm, col3_hbm, w3_out, degp_out, cntp_out,
            eid8, col8, w_v, ones_v, zb, deg_sh, cnt_sh,
            semI, semG, semS):
    cid, sid, wid = _ids()
    base = sid * SLC

    def _zb(i, _):
        zb[pl.ds(16 * i, 16)] = jnp.zeros((16,), _f32)
        return _
    lax.fori_loop(0, SLC // 16, _zb, None)
    for k in range(CK // 16):
        ones_v[pl.ds(16 * k, 16)] = jnp.ones((16,), _f32)
    pltpu.sync_copy(zb, deg_sh.at[pl.ds(base, SLC)])
    pltpu.sync_copy(zb, cnt_sh.at[pl.ds(base, SLC)])
    plsc.subcore_barrier()

    def _idx(p, j):
        return [
            pltpu.make_async_copy(eid3_hbm.at[wid, j], eid8.at[p], semI),
            pltpu.make_async_copy(col3_hbm.at[wid, j], col8.at[p], semI),
        ]

    def _gather(j, sg):
        return pltpu.make_async_copy(attr_hbm.at[eid8.at[jnp.bitwise_and(j, 7)]],
                                     w_v.at[j], sg)

    def _scats(j, ss):
        s = jnp.bitwise_and(j, 7)
        return [
            pltpu.make_async_copy(w_v.at[j], deg_sh.at[col8.at[s]], ss),
            pltpu.make_async_copy(ones_v, cnt_sh.at[col8.at[s]], ss),
        ]

    for j in range(6):
        for d in _idx(j, j):
            d.start()
    for d in _idx(0, 0):
        d.wait()
    for d in _idx(1, 1):
        d.wait()
    _gather(0, semG.at[0]).start()
    _gather(1, semG.at[1]).start()

    def _body(j, _):
        sg = jnp.bitwise_and(j, 1)

        @pl.when(j >= 2)
        def _():
            for d in _scats(j - 2, semS.at[sg]):
                d.wait()

        @pl.when(j + 6 < CH)
        def _():
            for d in _idx(jnp.bitwise_and(j + 6, 7), j + 6):
                d.start()

        _gather(j, semG.at[sg]).wait()
        for d in _scats(j, semS.at[sg]):
            d.start(add=True)

        @pl.when(j + 2 < CH)
        def _():
            for d in _idx(jnp.bitwise_and(j + 2, 7), j + 2):
                d.wait()
            _gather(j + 2, semG.at[sg]).start()
        return _

    lax.fori_loop(0, CH, _body, None)
    for d in _scats(CH - 2, semS.at[1]):
        d.wait()
    for d in _scats(CH - 1, semS.at[0]):
        d.wait()
    pltpu.sync_copy(w_v, w3_out.at[wid])
    plsc.subcore_barrier()
    pltpu.sync_copy(deg_sh.at[pl.ds(base, SLC)], degp_out.at[cid, pl.ds(base, SLC)])
    pltpu.sync_copy(cnt_sh.at[pl.ds(base, SLC)], cntp_out.at[cid, pl.ds(base, SLC)])


def _zero_acc(zb2, acc_sh, sid, sem):
    zr = zb2.shape[0]

    def _zb(i, _):
        for k in range(D // 16):
            zb2[i, pl.ds(16 * k, 16)] = jnp.zeros((16,), _f32)
        return _
    lax.fori_loop(0, zr, _zb, None)
    base = sid * SLC
    ds = []
    for t in range(SLC // zr):
        ds.append(pltpu.async_copy(zb2, acc_sh.at[pl.ds(base + t * zr, zr), :], sem))
    for d in ds:
        d.wait()


def _agg_pass(table_hbm, row3_hbm, col3_hbm, w3_hbm, out_ref,
              row8, col8, w8, buf4, zb2, acc_sh, semI, semG, semS):
    cid, sid, wid = _ids()
    base = sid * SLC
    _zero_acc(zb2, acc_sh, sid, semI)
    plsc.subcore_barrier()

    def _idx(p, j):
        ds = [
            pltpu.make_async_copy(row3_hbm.at[wid, j], row8.at[p], semI),
            pltpu.make_async_copy(col3_hbm.at[wid, j], col8.at[p], semI),
        ]
        if w8 is not None:
            ds.append(pltpu.make_async_copy(w3_hbm.at[wid, j], w8.at[p], semI))
        return ds

    def _gather(j, sg):
        return pltpu.make_async_copy(
            table_hbm.at[row8.at[jnp.bitwise_and(j, 7)]],
            buf4.at[jnp.bitwise_and(j, 3)], sg)

    def _scat(j, ss):
        return pltpu.make_async_copy(
            buf4.at[jnp.bitwise_and(j, 3)],
            acc_sh.at[col8.at[jnp.bitwise_and(j, 7)]], ss)

    def _scale(j):
        b = jnp.bitwise_and(j, 3)
        s = jnp.bitwise_and(j, 7)
        @plsc.parallel_loop(0, CK // 16)
        def _grp(k):
            wvec = w8[s, pl.ds(16 * k, 16)]
            for l in range(16):
                sc = wvec[l]
                bufe = buf4.at[b, 16 * k + l]
                vals = [bufe[pl.ds(16 * q, 16)] * sc for q in range(D // 16)]
                for q in range(D // 16):
                    bufe[pl.ds(16 * q, 16)] = vals[q]

    for j in range(6):
        for d in _idx(j, j):
            d.start()
    for d in _idx(0, 0):
        d.wait()
    for d in _idx(1, 1):
        d.wait()
    _gather(0, semG.at[0]).start()
    _gather(1, semG.at[1]).start()

    def _body(j, _):
        sg = jnp.bitwise_and(j, 1)

        @pl.when(j >= 2)
        def _():
            _scat(j - 2, semS.at[sg]).wait()

        @pl.when(j + 6 < CH)
        def _():
            for d in _idx(jnp.bitwise_and(j + 6, 7), j + 6):
                d.start()

        _gather(j, semG.at[sg]).wait()
        if w8 is not None:
            _scale(j)
        _scat(j, semS.at[sg]).start(add=True)

        @pl.when(j + 2 < CH)
        def _():
            for d in _idx(jnp.bitwise_and(j + 2, 7), j + 2):
                d.wait()
            _gather(j + 2, semG.at[sg]).start()
        return _

    lax.fori_loop(0, CH, _body, None)
    _scat(CH - 2, semS.at[1]).wait()
    _scat(CH - 1, semS.at[0]).wait()
    plsc.subcore_barrier()
    pltpu.sync_copy(acc_sh.at[pl.ds(base, SLC), :], out_ref.at[cid, pl.ds(base, SLC), :])


@functools.partial(
    pl.kernel,
    out_type=[jax.ShapeDtypeStruct((NC, N_ACC, D), _f32)],
    mesh=_mesh,
    scratch_types=[
        pltpu.VMEM((8, CK), jnp.int32),
        pltpu.VMEM((8, CK), jnp.int32),
        pltpu.VMEM((8, CK), _f32),
        pltpu.VMEM((4, CK, D), _f32),
        pltpu.VMEM((20, D), _f32),
        pltpu.VMEM_SHARED((N_ACC, D), _f32),
        pltpu.SemaphoreType.DMA,
        pltpu.SemaphoreType.DMA((2,)),
        pltpu.SemaphoreType.DMA((2,)),
    ],
)
def _pass_b(h2_hbm, row3_hbm, col3_hbm, w3_hbm, acc_out,
            row8, col8, w8, buf4, zb2, acc_sh, semI, semG, semS):
    _agg_pass(h2_hbm, row3_hbm, col3_hbm, w3_hbm, acc_out,
              row8, col8, w8, buf4, zb2, acc_sh, semI, semG, semS)


@functools.partial(
    pl.kernel,
    out_type=[jax.ShapeDtypeStruct((NC, N_ACC, D), _f32)],
    mesh=_mesh,
    scratch_types=[
        pltpu.VMEM((8, CK), jnp.int32),
        pltpu.VMEM((8, CK), jnp.int32),
        pltpu.VMEM((4, CK, D), _f32),
        pltpu.VMEM((20, D), _f32),
        pltpu.VMEM_SHARED((N_ACC, D), _f32),
        pltpu.SemaphoreType.DMA,
        pltpu.SemaphoreType.DMA((2,)),
        pltpu.SemaphoreType.DMA((2,)),
    ],
)
def _pass_c(gcn_hbm, row3_hbm, col3_hbm, sum_out,
            row8, col8, buf4, zb2, acc_sh, semI, semG, semS):
    _agg_pass(gcn_hbm, row3_hbm, col3_hbm, None, sum_out,
              row8, col8, None, buf4, zb2, acc_sh, semI, semG, semS)


_RB = 1000


def _tc12_body(x_ref, w_ref, degA_ref, degB_ref, h2_ref, hd_ref, dinv_ref):
    h = jnp.dot(x_ref[...], w_ref[...], preferred_element_type=_f32)
    deg = 1.0 + degA_ref[...] + degB_ref[...]
    dinv = lax.rsqrt(deg)
    h2_ref[...] = h * dinv
    hd_ref[...] = h / deg
    dinv_ref[...] = dinv


def _tc12(x_t, W_gcn, degA, degB):
    return pl.pallas_call(
        _tc12_body,
        grid=(N_T // _RB,),
        in_specs=[
            pl.BlockSpec((_RB, D), lambda i: (i, 0)),
            pl.BlockSpec((D, D), lambda i: (0, 0)),
            pl.BlockSpec((_RB, 1), lambda i: (i, 0)),
            pl.BlockSpec((_RB, 1), lambda i: (i, 0)),
        ],
        out_specs=[
            pl.BlockSpec((_RB, D), lambda i: (i, 0)),
            pl.BlockSpec((_RB, D), lambda i: (i, 0)),
            pl.BlockSpec((_RB, 1), lambda i: (i, 0)),
        ],
        out_shape=[
            jax.ShapeDtypeStruct((N_T, D), _f32),
            jax.ShapeDtypeStruct((N_T, D), _f32),
            jax.ShapeDtypeStruct((N_T, 1), _f32),
        ],
    )(x_t, W_gcn, degA, degB)


def _tc3_body(dinv_ref, accA_ref, accB_ref, hd_ref, bg_ref, gcn_ref):
    gcn_ref[...] = (dinv_ref[...] * (accA_ref[...] + accB_ref[...])
                    + hd_ref[...] + bg_ref[...])


def _tc3(dinv, accA, accB, hd, bg):
    return pl.pallas_call(
        _tc3_body,
        grid=(N_T // _RB,),
        in_specs=[
            pl.BlockSpec((_RB, 1), lambda i: (i, 0)),
            pl.BlockSpec((_RB, D), lambda i: (i, 0)),
            pl.BlockSpec((_RB, D), lambda i: (i, 0)),
            pl.BlockSpec((_RB, D), lambda i: (i, 0)),
            pl.BlockSpec((1, D), lambda i: (0, 0)),
        ],
        out_specs=pl.BlockSpec((_RB, D), lambda i: (i, 0)),
        out_shape=jax.ShapeDtypeStruct((N_T, D), _f32),
    )(dinv, accA, accB, hd, bg)


def _tc4_body(sA_ref, sB_ref, cntA_ref, cntB_ref, gcn_ref, wl_ref, wr_ref,
              bl_ref, out_ref):
    cnt = jnp.maximum(cntA_ref[...] + cntB_ref[...], 1.0)
    mean = (sA_ref[...] + sB_ref[...]) / cnt
    o = (jnp.dot(mean, wl_ref[...], preferred_element_type=_f32)
         + jnp.dot(gcn_ref[...], wr_ref[...], preferred_element_type=_f32)
         + bl_ref[...])
    ss = jnp.sum(o * o, axis=-1, keepdims=True)
    nrm = jnp.sqrt(jnp.maximum(ss, 1e-24))
    out_ref[...] = o / jnp.maximum(nrm, 1e-12)


def _tc4(sA, sB, cntA, cntB, gcn, WlT, WrT, bl):
    return pl.pallas_call(
        _tc4_body,
        grid=(N_T // _RB,),
        in_specs=[
            pl.BlockSpec((_RB, D), lambda i: (i, 0)),
            pl.BlockSpec((_RB, D), lambda i: (i, 0)),
            pl.BlockSpec((_RB, 1), lambda i: (i, 0)),
            pl.BlockSpec((_RB, 1), lambda i: (i, 0)),
            pl.BlockSpec((_RB, D), lambda i: (i, 0)),
            pl.BlockSpec((D, D), lambda i: (0, 0)),
            pl.BlockSpec((D, D), lambda i: (0, 0)),
            pl.BlockSpec((1, D), lambda i: (0, 0)),
        ],
        out_specs=pl.BlockSpec((_RB, D), lambda i: (i, 0)),
        out_shape=jax.ShapeDtypeStruct((N_T, D), _f32),
    )(sA, sB, cntA, cntB, gcn, WlT, WrT, bl)


def kernel(x, edge_index, e_id, attr, W_gcn, b_gcn, W_l, b_l, W_r):
    row3 = edge_index[0].reshape(NW, CH, CK)
    col3 = edge_index[1].reshape(NW, CH, CK)
    eid3 = e_id.reshape(NW, CH, CK)
    x_t = x[:N_T]

    w3, degp, cntp = _pass_a(attr, eid3, col3)
    degA = degp[0, :N_T, None]
    degB = degp[1, :N_T, None]
    h2, hd, dinv = _tc12(x_t, W_gcn, degA, degB)
    (accp,) = _pass_b(h2, row3, col3, w3)
    gcn = _tc3(dinv, accp[0, :N_T], accp[1, :N_T], hd, b_gcn[None, :])
    (sump,) = _pass_c(gcn, row3, col3)
    out = _tc4(sump[0, :N_T], sump[1, :N_T], cntp[0, :N_T, None],
               cntp[1, :N_T, None], gcn, W_l.T, W_r.T, b_l[None, :])
    return out


# speedup vs baseline: 30.4200x; 1.0485x over previous
import functools

import jax
import jax.numpy as jnp
from jax import lax
from jax.experimental import pallas as pl
from jax.experimental.pallas import tpu as pltpu
from jax.experimental.pallas import tpu_sc as plsc

N_T = 10000
D = 128
E = 320000
NC = 2
NS = 16
NW = NC * NS
CK = 80
CH = 125
N_ACC = 10240
SLC = N_ACC // NS

_mesh = plsc.VectorSubcoreMesh(core_axis_name="c", subcore_axis_name="s")
_f32 = jnp.float32


def _ids():
    cid = lax.axis_index("c")
    sid = lax.axis_index("s")
    return cid, sid, sid * NC + cid


@functools.partial(
    pl.kernel,
    out_type=[
        jax.ShapeDtypeStruct((NW, CH, CK), _f32),
        jax.ShapeDtypeStruct((NC, N_ACC), _f32),
        jax.ShapeDtypeStruct((NC, N_ACC), _f32),
    ],
    mesh=_mesh,
    scratch_types=[
        pltpu.VMEM((8, CK), jnp.int32),
        pltpu.VMEM((8, CK), jnp.int32),
        pltpu.VMEM((CH, CK), _f32),
        pltpu.VMEM((CK,), _f32),
        pltpu.VMEM((SLC,), _f32),
        pltpu.VMEM_SHARED((N_ACC,), _f32),
        pltpu.VMEM_SHARED((N_ACC,), _f32),
        pltpu.SemaphoreType.DMA,
        pltpu.SemaphoreType.DMA((2,)),
        pltpu.SemaphoreType.DMA((2,)),
    ],
)
def _pass_a(attr_hbm, eid3_hbm, col3_hbm, w3_out, degp_out, cntp_out,
            eid8, col8, w_v, ones_v, zb, deg_sh, cnt_sh,
            semI, semG, semS):
    cid, sid, wid = _ids()
    base = sid * SLC

    def _zb(i, _):
        zb[pl.ds(16 * i, 16)] = jnp.zeros((16,), _f32)
        return _
    lax.fori_loop(0, SLC // 16, _zb, None)
    for k in range(CK // 16):
        ones_v[pl.ds(16 * k, 16)] = jnp.ones((16,), _f32)
    pltpu.sync_copy(zb, deg_sh.at[pl.ds(base, SLC)])
    pltpu.sync_copy(zb, cnt_sh.at[pl.ds(base, SLC)])
    plsc.subcore_barrier()

    def _idx(p, j):
        return [
            pltpu.make_async_copy(eid3_hbm.at[wid, j], eid8.at[p], semI),
            pltpu.make_async_copy(col3_hbm.at[wid, j], col8.at[p], semI),
        ]

    def _gather(j, sg):
        return pltpu.make_async_copy(attr_hbm.at[eid8.at[jnp.bitwise_and(j, 7)]],
                                     w_v.at[j], sg)

    def _scats(j, ss):
        s = jnp.bitwise_and(j, 7)
        return [
            pltpu.make_async_copy(w_v.at[j], deg_sh.at[col8.at[s]], ss),
            pltpu.make_async_copy(ones_v, cnt_sh.at[col8.at[s]], ss),
        ]

    for j in range(6):
        for d in _idx(j, j):
            d.start()
    for d in _idx(0, 0):
        d.wait()
    for d in _idx(1, 1):
        d.wait()
    _gather(0, semG.at[0]).start()
    _gather(1, semG.at[1]).start()

    def _body(j, _):
        sg = jnp.bitwise_and(j, 1)

        @pl.when(j >= 2)
        def _():
            for d in _scats(j - 2, semS.at[sg]):
                d.wait()

        @pl.when(j + 6 < CH)
        def _():
            for d in _idx(jnp.bitwise_and(j + 6, 7), j + 6):
                d.start()

        _gather(j, semG.at[sg]).wait()
        for d in _scats(j, semS.at[sg]):
            d.start(add=True)

        @pl.when(j + 2 < CH)
        def _():
            for d in _idx(jnp.bitwise_and(j + 2, 7), j + 2):
                d.wait()
            _gather(j + 2, semG.at[sg]).start()
        return _

    lax.fori_loop(0, CH, _body, None)
    for d in _scats(CH - 2, semS.at[1]):
        d.wait()
    for d in _scats(CH - 1, semS.at[0]):
        d.wait()
    pltpu.sync_copy(w_v, w3_out.at[wid])
    plsc.subcore_barrier()
    pltpu.sync_copy(deg_sh.at[pl.ds(base, SLC)], degp_out.at[cid, pl.ds(base, SLC)])
    pltpu.sync_copy(cnt_sh.at[pl.ds(base, SLC)], cntp_out.at[cid, pl.ds(base, SLC)])


def _zero_acc(zb2, acc_sh, sid, sem):
    zr = zb2.shape[0]

    def _zb(i, _):
        for k in range(D // 16):
            zb2[i, pl.ds(16 * k, 16)] = jnp.zeros((16,), _f32)
        return _
    lax.fori_loop(0, zr, _zb, None)
    base = sid * SLC
    ds = []
    for t in range(SLC // zr):
        ds.append(pltpu.async_copy(zb2, acc_sh.at[pl.ds(base + t * zr, zr), :], sem))
    for d in ds:
        d.wait()


def _agg_pass(table_hbm, row3_hbm, col3_hbm, w3_hbm, out_ref,
              row8, col8, w8, buf4, zb2, acc_sh, semI, semG, semS):
    cid, sid, wid = _ids()
    base = sid * SLC
    _zero_acc(zb2, acc_sh, sid, semI)
    plsc.subcore_barrier()

    def _idx(p, j):
        ds = [
            pltpu.make_async_copy(row3_hbm.at[wid, j], row8.at[p], semI),
            pltpu.make_async_copy(col3_hbm.at[wid, j], col8.at[p], semI),
        ]
        if w8 is not None:
            ds.append(pltpu.make_async_copy(w3_hbm.at[wid, j], w8.at[p], semI))
        return ds

    def _gather(j, sg):
        return pltpu.make_async_copy(
            table_hbm.at[row8.at[jnp.bitwise_and(j, 7)]],
            buf4.at[jnp.bitwise_and(j, 3)], sg)

    def _scat(j, ss):
        return pltpu.make_async_copy(
            buf4.at[jnp.bitwise_and(j, 3)],
            acc_sh.at[col8.at[jnp.bitwise_and(j, 7)]], ss)

    def _scale(j):
        b = jnp.bitwise_and(j, 3)
        s = jnp.bitwise_and(j, 7)
        @plsc.parallel_loop(0, CK // 16)
        def _grp(k):
            wvec = w8[s, pl.ds(16 * k, 16)]
            for l in range(16):
                sc = wvec[l]
                bufe = buf4.at[b, 16 * k + l]
                vals = [bufe[pl.ds(16 * q, 16)] * sc for q in range(D // 16)]
                for q in range(D // 16):
                    bufe[pl.ds(16 * q, 16)] = vals[q]

    for j in range(6):
        for d in _idx(j, j):
            d.start()
    for d in _idx(0, 0):
        d.wait()
    for d in _idx(1, 1):
        d.wait()
    _gather(0, semG.at[0]).start()
    _gather(1, semG.at[1]).start()

    def _body(j, _):
        sg = jnp.bitwise_and(j, 1)

        @pl.when(j >= 2)
        def _():
            _scat(j - 2, semS.at[sg]).wait()

        @pl.when(j + 6 < CH)
        def _():
            for d in _idx(jnp.bitwise_and(j + 6, 7), j + 6):
                d.start()

        _gather(j, semG.at[sg]).wait()
        if w8 is not None:
            _scale(j)
        _scat(j, semS.at[sg]).start(add=True)

        @pl.when(j + 2 < CH)
        def _():
            for d in _idx(jnp.bitwise_and(j + 2, 7), j + 2):
                d.wait()
            _gather(j + 2, semG.at[sg]).start()
        return _

    lax.fori_loop(0, CH, _body, None)
    _scat(CH - 2, semS.at[1]).wait()
    _scat(CH - 1, semS.at[0]).wait()
    plsc.subcore_barrier()
    pltpu.sync_copy(acc_sh.at[pl.ds(base, SLC), :], out_ref.at[cid, pl.ds(base, SLC), :])


@functools.partial(
    pl.kernel,
    out_type=[jax.ShapeDtypeStruct((NC, N_ACC, D), _f32)],
    mesh=_mesh,
    scratch_types=[
        pltpu.VMEM((8, CK), jnp.int32),
        pltpu.VMEM((8, CK), jnp.int32),
        pltpu.VMEM((8, CK), _f32),
        pltpu.VMEM((4, CK, D), _f32),
        pltpu.VMEM((20, D), _f32),
        pltpu.VMEM_SHARED((N_ACC, D), _f32),
        pltpu.SemaphoreType.DMA,
        pltpu.SemaphoreType.DMA((2,)),
        pltpu.SemaphoreType.DMA((2,)),
    ],
)
def _pass_b(h2_hbm, row3_hbm, col3_hbm, w3_hbm, acc_out,
            row8, col8, w8, buf4, zb2, acc_sh, semI, semG, semS):
    _agg_pass(h2_hbm, row3_hbm, col3_hbm, w3_hbm, acc_out,
              row8, col8, w8, buf4, zb2, acc_sh, semI, semG, semS)


@functools.partial(
    pl.kernel,
    out_type=[jax.ShapeDtypeStruct((NC, N_ACC, D), _f32)],
    mesh=_mesh,
    scratch_types=[
        pltpu.VMEM((8, CK), jnp.int32),
        pltpu.VMEM((8, CK), jnp.int32),
        pltpu.VMEM((4, CK, D), _f32),
        pltpu.VMEM((20, D), _f32),
        pltpu.VMEM_SHARED((N_ACC, D), _f32),
        pltpu.SemaphoreType.DMA,
        pltpu.SemaphoreType.DMA((2,)),
        pltpu.SemaphoreType.DMA((2,)),
    ],
)
def _pass_c(gcn_hbm, row3_hbm, col3_hbm, sum_out,
            row8, col8, buf4, zb2, acc_sh, semI, semG, semS):
    _agg_pass(gcn_hbm, row3_hbm, col3_hbm, None, sum_out,
              row8, col8, None, buf4, zb2, acc_sh, semI, semG, semS)


_RB = 1000


def _tc12_body(x_ref, w_ref, degA_ref, degB_ref, h2_ref, hd_ref, dinv_ref):
    h = jnp.dot(x_ref[...], w_ref[...], preferred_element_type=_f32)
    deg = 1.0 + degA_ref[0] + degB_ref[0]
    dinv = lax.rsqrt(deg)
    h2_ref[...] = h * dinv
    hd_ref[...] = h / deg
    dinv_ref[...] = dinv


def _tc12(x, W_gcn, degp3):
    return pl.pallas_call(
        _tc12_body,
        grid=(N_T // _RB,),
        in_specs=[
            pl.BlockSpec((_RB, D), lambda i: (i, 0)),
            pl.BlockSpec((D, D), lambda i: (0, 0)),
            pl.BlockSpec((1, _RB, 1), lambda i: (0, i, 0)),
            pl.BlockSpec((1, _RB, 1), lambda i: (1, i, 0)),
        ],
        out_specs=[
            pl.BlockSpec((_RB, D), lambda i: (i, 0)),
            pl.BlockSpec((_RB, D), lambda i: (i, 0)),
            pl.BlockSpec((_RB, 1), lambda i: (i, 0)),
        ],
        out_shape=[
            jax.ShapeDtypeStruct((N_T, D), _f32),
            jax.ShapeDtypeStruct((N_T, D), _f32),
            jax.ShapeDtypeStruct((N_T, 1), _f32),
        ],
    )(x, W_gcn, degp3, degp3)


def _tc3_body(dinv_ref, accA_ref, accB_ref, hd_ref, bg_ref, gcn_ref):
    gcn_ref[...] = (dinv_ref[...] * (accA_ref[0] + accB_ref[0])
                    + hd_ref[...] + bg_ref[...])


def _tc3(dinv, accp, hd, bg):
    return pl.pallas_call(
        _tc3_body,
        grid=(N_T // _RB,),
        in_specs=[
            pl.BlockSpec((_RB, 1), lambda i: (i, 0)),
            pl.BlockSpec((1, _RB, D), lambda i: (0, i, 0)),
            pl.BlockSpec((1, _RB, D), lambda i: (1, i, 0)),
            pl.BlockSpec((_RB, D), lambda i: (i, 0)),
            pl.BlockSpec((1, D), lambda i: (0, 0)),
        ],
        out_specs=pl.BlockSpec((_RB, D), lambda i: (i, 0)),
        out_shape=jax.ShapeDtypeStruct((N_T, D), _f32),
    )(dinv, accp, accp, hd, bg)


def _tc4_body(sA_ref, sB_ref, cntA_ref, cntB_ref, gcn_ref, wl_ref, wr_ref,
              bl_ref, out_ref):
    cnt = jnp.maximum(cntA_ref[0] + cntB_ref[0], 1.0)
    mean = (sA_ref[0] + sB_ref[0]) / cnt
    o = (jnp.dot(mean, wl_ref[...], preferred_element_type=_f32)
         + jnp.dot(gcn_ref[...], wr_ref[...], preferred_element_type=_f32)
         + bl_ref[...])
    ss = jnp.sum(o * o, axis=-1, keepdims=True)
    nrm = jnp.sqrt(jnp.maximum(ss, 1e-24))
    out_ref[...] = o / jnp.maximum(nrm, 1e-12)


def _tc4(sump, cntp3, gcn, WlT, WrT, bl):
    return pl.pallas_call(
        _tc4_body,
        grid=(N_T // _RB,),
        in_specs=[
            pl.BlockSpec((1, _RB, D), lambda i: (0, i, 0)),
            pl.BlockSpec((1, _RB, D), lambda i: (1, i, 0)),
            pl.BlockSpec((1, _RB, 1), lambda i: (0, i, 0)),
            pl.BlockSpec((1, _RB, 1), lambda i: (1, i, 0)),
            pl.BlockSpec((_RB, D), lambda i: (i, 0)),
            pl.BlockSpec((D, D), lambda i: (0, 0)),
            pl.BlockSpec((D, D), lambda i: (0, 0)),
            pl.BlockSpec((1, D), lambda i: (0, 0)),
        ],
        out_specs=pl.BlockSpec((_RB, D), lambda i: (i, 0)),
        out_shape=jax.ShapeDtypeStruct((N_T, D), _f32),
    )(sump, sump, cntp3, cntp3, gcn, WlT, WrT, bl)


def kernel(x, edge_index, e_id, attr, W_gcn, b_gcn, W_l, b_l, W_r):
    row3 = edge_index[0].reshape(NW, CH, CK)
    col3 = edge_index[1].reshape(NW, CH, CK)
    eid3 = e_id.reshape(NW, CH, CK)

    w3, degp, cntp = _pass_a(attr, eid3, col3)
    h2, hd, dinv = _tc12(x, W_gcn, degp[:, :, None])
    (accp,) = _pass_b(h2, row3, col3, w3)
    gcn = _tc3(dinv, accp, hd, b_gcn[None, :])
    (sump,) = _pass_c(gcn, row3, col3)
    out = _tc4(sump, cntp[:, :, None], gcn, W_l.T, W_r.T, b_l[None, :])
    return out


# speedup vs baseline: 30.7013x; 1.0092x over previous
import functools

import jax
import jax.numpy as jnp
from jax import lax
from jax.experimental import pallas as pl
from jax.experimental.pallas import tpu as pltpu
from jax.experimental.pallas import tpu_sc as plsc

N_T = 10000
D = 128
E = 320000
NC = 2
NS = 16
NW = NC * NS
CK = 80
CH = 125
N_ACC = 10240
SLC = N_ACC // NS

_mesh = plsc.VectorSubcoreMesh(core_axis_name="c", subcore_axis_name="s")
_f32 = jnp.float32


def _ids():
    cid = lax.axis_index("c")
    sid = lax.axis_index("s")
    return cid, sid, sid * NC + cid


@functools.partial(
    pl.kernel,
    out_type=[
        jax.ShapeDtypeStruct((NW, CH, CK), _f32),
        jax.ShapeDtypeStruct((NC, N_ACC), _f32),
        jax.ShapeDtypeStruct((NC, N_ACC), _f32),
    ],
    mesh=_mesh,
    scratch_types=[
        pltpu.VMEM((8, CK), jnp.int32),
        pltpu.VMEM((8, CK), jnp.int32),
        pltpu.VMEM((CH, CK), _f32),
        pltpu.VMEM((CK,), _f32),
        pltpu.VMEM((SLC,), _f32),
        pltpu.VMEM_SHARED((N_ACC,), _f32),
        pltpu.VMEM_SHARED((N_ACC,), _f32),
        pltpu.SemaphoreType.DMA,
        pltpu.SemaphoreType.DMA((2,)),
        pltpu.SemaphoreType.DMA((2,)),
    ],
)
def _pass_a(attr_hbm, eid_hbm, col_hbm, w3_out, degp_out, cntp_out,
            eid8, col8, w_v, ones_v, zb, deg_sh, cnt_sh,
            semI, semG, semS):
    cid, sid, wid = _ids()
    base = sid * SLC

    def _zb(i, _):
        zb[pl.ds(16 * i, 16)] = jnp.zeros((16,), _f32)
        return _
    lax.fori_loop(0, SLC // 16, _zb, None)
    for k in range(CK // 16):
        ones_v[pl.ds(16 * k, 16)] = jnp.ones((16,), _f32)
    pltpu.sync_copy(zb, deg_sh.at[pl.ds(base, SLC)])
    pltpu.sync_copy(zb, cnt_sh.at[pl.ds(base, SLC)])
    plsc.subcore_barrier()

    def _idx(p, j):
        off = wid * (CH * CK) + j * CK
        return [
            pltpu.make_async_copy(eid_hbm.at[pl.ds(off, CK)], eid8.at[p], semI),
            pltpu.make_async_copy(col_hbm.at[pl.ds(off, CK)], col8.at[p], semI),
        ]

    def _gather(j, sg):
        return pltpu.make_async_copy(attr_hbm.at[eid8.at[jnp.bitwise_and(j, 7)]],
                                     w_v.at[j], sg)

    def _scats(j, ss):
        s = jnp.bitwise_and(j, 7)
        return [
            pltpu.make_async_copy(w_v.at[j], deg_sh.at[col8.at[s]], ss),
            pltpu.make_async_copy(ones_v, cnt_sh.at[col8.at[s]], ss),
        ]

    for j in range(6):
        for d in _idx(j, j):
            d.start()
    for d in _idx(0, 0):
        d.wait()
    for d in _idx(1, 1):
        d.wait()
    _gather(0, semG.at[0]).start()
    _gather(1, semG.at[1]).start()

    def _body(j, _):
        sg = jnp.bitwise_and(j, 1)

        @pl.when(j >= 2)
        def _():
            for d in _scats(j - 2, semS.at[sg]):
                d.wait()

        @pl.when(j + 6 < CH)
        def _():
            for d in _idx(jnp.bitwise_and(j + 6, 7), j + 6):
                d.start()

        _gather(j, semG.at[sg]).wait()
        for d in _scats(j, semS.at[sg]):
            d.start(add=True)

        @pl.when(j + 2 < CH)
        def _():
            for d in _idx(jnp.bitwise_and(j + 2, 7), j + 2):
                d.wait()
            _gather(j + 2, semG.at[sg]).start()
        return _

    lax.fori_loop(0, CH, _body, None)
    for d in _scats(CH - 2, semS.at[1]):
        d.wait()
    for d in _scats(CH - 1, semS.at[0]):
        d.wait()
    pltpu.sync_copy(w_v, w3_out.at[wid])
    plsc.subcore_barrier()
    pltpu.sync_copy(deg_sh.at[pl.ds(base, SLC)], degp_out.at[cid, pl.ds(base, SLC)])
    pltpu.sync_copy(cnt_sh.at[pl.ds(base, SLC)], cntp_out.at[cid, pl.ds(base, SLC)])


def _zero_acc(zb2, acc_sh, sid, sem):
    zr = zb2.shape[0]

    def _zb(i, _):
        for k in range(D // 16):
            zb2[i, pl.ds(16 * k, 16)] = jnp.zeros((16,), _f32)
        return _
    lax.fori_loop(0, zr, _zb, None)
    base = sid * SLC
    ds = []
    for t in range(SLC // zr):
        ds.append(pltpu.async_copy(zb2, acc_sh.at[pl.ds(base + t * zr, zr), :], sem))
    for d in ds:
        d.wait()


def _agg_pass(table_hbm, row_hbm, col_hbm, w3_hbm, out_ref,
              row8, col8, w8, buf4, zb2, acc_sh, semI, semG, semS, seed):
    cid, sid, wid = _ids()
    base = sid * SLC
    if seed:
        @pl.when(cid == 0)
        def _():
            pltpu.sync_copy(table_hbm.at[pl.ds(base, SLC), :],
                            acc_sh.at[pl.ds(base, SLC), :])

        @pl.when(cid != 0)
        def _():
            _zero_acc(zb2, acc_sh, sid, semI)
    else:
        _zero_acc(zb2, acc_sh, sid, semI)
    plsc.subcore_barrier()

    def _idx(p, j):
        off = wid * (CH * CK) + j * CK
        ds = [
            pltpu.make_async_copy(row_hbm.at[pl.ds(off, CK)], row8.at[p], semI),
            pltpu.make_async_copy(col_hbm.at[pl.ds(off, CK)], col8.at[p], semI),
        ]
        if w8 is not None:
            ds.append(pltpu.make_async_copy(w3_hbm.at[wid, j], w8.at[p], semI))
        return ds

    def _gather(j, sg):
        return pltpu.make_async_copy(
            table_hbm.at[row8.at[jnp.bitwise_and(j, 7)]],
            buf4.at[jnp.bitwise_and(j, 3)], sg)

    def _scat(j, ss):
        return pltpu.make_async_copy(
            buf4.at[jnp.bitwise_and(j, 3)],
            acc_sh.at[col8.at[jnp.bitwise_and(j, 7)]], ss)

    def _scale(j):
        b = jnp.bitwise_and(j, 3)
        s = jnp.bitwise_and(j, 7)
        @plsc.parallel_loop(0, CK // 16)
        def _grp(k):
            wvec = w8[s, pl.ds(16 * k, 16)]
            for l in range(16):
                sc = wvec[l]
                bufe = buf4.at[b, 16 * k + l]
                vals = [bufe[pl.ds(16 * q, 16)] * sc for q in range(D // 16)]
                for q in range(D // 16):
                    bufe[pl.ds(16 * q, 16)] = vals[q]

    for j in range(6):
        for d in _idx(j, j):
            d.start()
    for d in _idx(0, 0):
        d.wait()
    for d in _idx(1, 1):
        d.wait()
    _gather(0, semG.at[0]).start()
    _gather(1, semG.at[1]).start()

    def _body(j, _):
        sg = jnp.bitwise_and(j, 1)

        @pl.when(j >= 2)
        def _():
            _scat(j - 2, semS.at[sg]).wait()

        @pl.when(j + 6 < CH)
        def _():
            for d in _idx(jnp.bitwise_and(j + 6, 7), j + 6):
                d.start()

        _gather(j, semG.at[sg]).wait()
        if w8 is not None:
            _scale(j)
        _scat(j, semS.at[sg]).start(add=True)

        @pl.when(j + 2 < CH)
        def _():
            for d in _idx(jnp.bitwise_and(j + 2, 7), j + 2):
                d.wait()
            _gather(j + 2, semG.at[sg]).start()
        return _

    lax.fori_loop(0, CH, _body, None)
    _scat(CH - 2, semS.at[1]).wait()
    _scat(CH - 1, semS.at[0]).wait()
    plsc.subcore_barrier()
    pltpu.sync_copy(acc_sh.at[pl.ds(base, SLC), :], out_ref.at[cid, pl.ds(base, SLC), :])


@functools.partial(
    pl.kernel,
    out_type=[jax.ShapeDtypeStruct((NC, N_ACC, D), _f32)],
    mesh=_mesh,
    scratch_types=[
        pltpu.VMEM((8, CK), jnp.int32),
        pltpu.VMEM((8, CK), jnp.int32),
        pltpu.VMEM((8, CK), _f32),
        pltpu.VMEM((4, CK, D), _f32),
        pltpu.VMEM((20, D), _f32),
        pltpu.VMEM_SHARED((N_ACC, D), _f32),
        pltpu.SemaphoreType.DMA,
        pltpu.SemaphoreType.DMA((2,)),
        pltpu.SemaphoreType.DMA((2,)),
    ],
)
def _pass_b(h2_hbm, row_hbm, col_hbm, w3_hbm, acc_out,
            row8, col8, w8, buf4, zb2, acc_sh, semI, semG, semS):
    _agg_pass(h2_hbm, row_hbm, col_hbm, w3_hbm, acc_out,
              row8, col8, w8, buf4, zb2, acc_sh, semI, semG, semS, True)


@functools.partial(
    pl.kernel,
    out_type=[jax.ShapeDtypeStruct((NC, N_ACC, D), _f32)],
    mesh=_mesh,
    scratch_types=[
        pltpu.VMEM((8, CK), jnp.int32),
        pltpu.VMEM((8, CK), jnp.int32),
        pltpu.VMEM((4, CK, D), _f32),
        pltpu.VMEM((20, D), _f32),
        pltpu.VMEM_SHARED((N_ACC, D), _f32),
        pltpu.SemaphoreType.DMA,
        pltpu.SemaphoreType.DMA((2,)),
        pltpu.SemaphoreType.DMA((2,)),
    ],
)
def _pass_c(gcn_hbm, row_hbm, col_hbm, sum_out,
            row8, col8, buf4, zb2, acc_sh, semI, semG, semS):
    _agg_pass(gcn_hbm, row_hbm, col_hbm, None, sum_out,
              row8, col8, None, buf4, zb2, acc_sh, semI, semG, semS, False)


_RB = 1000


def _tc12_body(x_ref, w_ref, degA_ref, degB_ref, h2_ref, dinv_ref):
    h = jnp.dot(x_ref[...], w_ref[...], preferred_element_type=_f32)
    deg = 1.0 + degA_ref[0] + degB_ref[0]
    dinv = lax.rsqrt(deg)
    h2_ref[...] = h * dinv
    dinv_ref[...] = dinv


_RB12 = 1024


def _tc12(x, W_gcn, degp3):
    return pl.pallas_call(
        _tc12_body,
        grid=(N_ACC // _RB12,),
        in_specs=[
            pl.BlockSpec((_RB12, D), lambda i: (i, 0)),
            pl.BlockSpec((D, D), lambda i: (0, 0)),
            pl.BlockSpec((1, _RB12, 1), lambda i: (0, i, 0)),
            pl.BlockSpec((1, _RB12, 1), lambda i: (1, i, 0)),
        ],
        out_specs=[
            pl.BlockSpec((_RB12, D), lambda i: (i, 0)),
            pl.BlockSpec((_RB12, 1), lambda i: (i, 0)),
        ],
        out_shape=[
            jax.ShapeDtypeStruct((N_ACC, D), _f32),
            jax.ShapeDtypeStruct((N_ACC, 1), _f32),
        ],
    )(x, W_gcn, degp3, degp3)


def _tc3_body(dinv_ref, accA_ref, accB_ref, bg_ref, gcn_ref):
    gcn_ref[...] = (dinv_ref[...] * (accA_ref[0] + accB_ref[0])
                    + bg_ref[...])


def _tc3(dinv, accp, bg):
    return pl.pallas_call(
        _tc3_body,
        grid=(N_T // _RB,),
        in_specs=[
            pl.BlockSpec((_RB, 1), lambda i: (i, 0)),
            pl.BlockSpec((1, _RB, D), lambda i: (0, i, 0)),
            pl.BlockSpec((1, _RB, D), lambda i: (1, i, 0)),
            pl.BlockSpec((1, D), lambda i: (0, 0)),
        ],
        out_specs=pl.BlockSpec((_RB, D), lambda i: (i, 0)),
        out_shape=jax.ShapeDtypeStruct((N_T, D), _f32),
    )(dinv, accp, accp, bg)


def _tc4_body(sA_ref, sB_ref, cntA_ref, cntB_ref, gcn_ref, wl_ref, wr_ref,
              bl_ref, out_ref):
    cnt = jnp.maximum(cntA_ref[0] + cntB_ref[0], 1.0)
    mean = (sA_ref[0] + sB_ref[0]) / cnt
    o = (lax.dot_general(mean, wl_ref[...], (((1,), (1,)), ((), ())),
                         preferred_element_type=_f32)
         + lax.dot_general(gcn_ref[...], wr_ref[...], (((1,), (1,)), ((), ())),
                           preferred_element_type=_f32)
         + bl_ref[...])
    ss = jnp.sum(o * o, axis=-1, keepdims=True)
    nrm = jnp.sqrt(jnp.maximum(ss, 1e-24))
    out_ref[...] = o / jnp.maximum(nrm, 1e-12)


def _tc4(sump, cntp3, gcn, Wl, Wr, bl):
    return pl.pallas_call(
        _tc4_body,
        grid=(N_T // _RB,),
        in_specs=[
            pl.BlockSpec((1, _RB, D), lambda i: (0, i, 0)),
            pl.BlockSpec((1, _RB, D), lambda i: (1, i, 0)),
            pl.BlockSpec((1, _RB, 1), lambda i: (0, i, 0)),
            pl.BlockSpec((1, _RB, 1), lambda i: (1, i, 0)),
            pl.BlockSpec((_RB, D), lambda i: (i, 0)),
            pl.BlockSpec((D, D), lambda i: (0, 0)),
            pl.BlockSpec((D, D), lambda i: (0, 0)),
            pl.BlockSpec((1, D), lambda i: (0, 0)),
        ],
        out_specs=pl.BlockSpec((_RB, D), lambda i: (i, 0)),
        out_shape=jax.ShapeDtypeStruct((N_T, D), _f32),
    )(sump, sump, cntp3, cntp3, gcn, Wl, Wr, bl)


def kernel(x, edge_index, e_id, attr, W_gcn, b_gcn, W_l, b_l, W_r):
    row = edge_index[0]
    col = edge_index[1]
    w3, degp, cntp = _pass_a(attr, e_id, col)
    h2, dinv = _tc12(x, W_gcn, degp[:, :, None])
    (accp,) = _pass_b(h2, row, col, w3)
    gcn = _tc3(dinv, accp, b_gcn[None, :])
    (sump,) = _pass_c(gcn, row, col)
    out = _tc4(sump, cntp[:, :, None], gcn, W_l, W_r, b_l[None, :])
    return out


# speedup vs baseline: 31.7428x; 1.0339x over previous
import functools

import jax
import jax.numpy as jnp
from jax import lax
from jax.experimental import pallas as pl
from jax.experimental.pallas import tpu as pltpu
from jax.experimental.pallas import tpu_sc as plsc

N_T = 10000
D = 128
E = 320000
NC = 2
NS = 16
NW = NC * NS
CK = 80
CH = 125
N_ACC = 10240
SLC = N_ACC // NS

_mesh = plsc.VectorSubcoreMesh(core_axis_name="c", subcore_axis_name="s")
_f32 = jnp.float32


def _ids():
    cid = lax.axis_index("c")
    sid = lax.axis_index("s")
    return cid, sid, sid * NC + cid


@functools.partial(
    pl.kernel,
    out_type=[
        jax.ShapeDtypeStruct((NW, CH, CK), _f32),
        jax.ShapeDtypeStruct((NC, N_ACC), _f32),
        jax.ShapeDtypeStruct((NC, N_ACC), _f32),
    ],
    mesh=_mesh,
    scratch_types=[
        pltpu.VMEM((8, CK), jnp.int32),
        pltpu.VMEM((8, CK), jnp.int32),
        pltpu.VMEM((CH, CK), _f32),
        pltpu.VMEM((CK,), _f32),
        pltpu.VMEM((SLC,), _f32),
        pltpu.VMEM_SHARED((N_ACC,), _f32),
        pltpu.VMEM_SHARED((N_ACC,), _f32),
        pltpu.SemaphoreType.DMA,
        pltpu.SemaphoreType.DMA((2,)),
        pltpu.SemaphoreType.DMA((2,)),
    ],
)
def _pass_a(attr_hbm, eid_hbm, ei_hbm, w3_out, degp_out, cntp_out,
            eid8, col8, w_v, ones_v, zb, deg_sh, cnt_sh,
            semI, semG, semS):
    cid, sid, wid = _ids()
    base = sid * SLC

    def _zb(i, _):
        zb[pl.ds(16 * i, 16)] = jnp.zeros((16,), _f32)
        return _
    lax.fori_loop(0, SLC // 16, _zb, None)
    for k in range(CK // 16):
        ones_v[pl.ds(16 * k, 16)] = jnp.ones((16,), _f32)
    pltpu.sync_copy(zb, deg_sh.at[pl.ds(base, SLC)])
    pltpu.sync_copy(zb, cnt_sh.at[pl.ds(base, SLC)])
    plsc.subcore_barrier()

    def _idx(p, j):
        off = wid * (CH * CK) + j * CK
        return [
            pltpu.make_async_copy(eid_hbm.at[pl.ds(off, CK)], eid8.at[p], semI),
            pltpu.make_async_copy(ei_hbm.at[pl.ds(E + off, CK)], col8.at[p], semI),
        ]

    def _gather(j, sg):
        return pltpu.make_async_copy(attr_hbm.at[eid8.at[jnp.bitwise_and(j, 7)]],
                                     w_v.at[j], sg)

    def _scats(j, ss):
        s = jnp.bitwise_and(j, 7)
        return [
            pltpu.make_async_copy(w_v.at[j], deg_sh.at[col8.at[s]], ss),
            pltpu.make_async_copy(ones_v, cnt_sh.at[col8.at[s]], ss),
        ]

    for j in range(6):
        for d in _idx(j, j):
            d.start()
    for d in _idx(0, 0):
        d.wait()
    for d in _idx(1, 1):
        d.wait()
    _gather(0, semG.at[0]).start()
    _gather(1, semG.at[1]).start()

    def _body(j, _):
        sg = jnp.bitwise_and(j, 1)

        @pl.when(j >= 2)
        def _():
            for d in _scats(j - 2, semS.at[sg]):
                d.wait()

        @pl.when(j + 6 < CH)
        def _():
            for d in _idx(jnp.bitwise_and(j + 6, 7), j + 6):
                d.start()

        _gather(j, semG.at[sg]).wait()
        for d in _scats(j, semS.at[sg]):
            d.start(add=True)

        @pl.when(j + 2 < CH)
        def _():
            for d in _idx(jnp.bitwise_and(j + 2, 7), j + 2):
                d.wait()
            _gather(j + 2, semG.at[sg]).start()
        return _

    lax.fori_loop(0, CH, _body, None)
    for d in _scats(CH - 2, semS.at[1]):
        d.wait()
    for d in _scats(CH - 1, semS.at[0]):
        d.wait()
    pltpu.sync_copy(w_v, w3_out.at[wid])
    plsc.subcore_barrier()
    pltpu.sync_copy(deg_sh.at[pl.ds(base, SLC)], degp_out.at[cid, pl.ds(base, SLC)])
    pltpu.sync_copy(cnt_sh.at[pl.ds(base, SLC)], cntp_out.at[cid, pl.ds(base, SLC)])


def _zero_acc(zb2, acc_sh, sid, sem):
    zr = zb2.shape[0]

    def _zb(i, _):
        for k in range(D // 16):
            zb2[i, pl.ds(16 * k, 16)] = jnp.zeros((16,), _f32)
        return _
    lax.fori_loop(0, zr, _zb, None)
    base = sid * SLC
    ds = []
    for t in range(SLC // zr):
        ds.append(pltpu.async_copy(zb2, acc_sh.at[pl.ds(base + t * zr, zr), :], sem))
    for d in ds:
        d.wait()


def _agg_pass(table_hbm, ei_hbm, w3_hbm, out_ref,
              row8, col8, w8, buf4, zb2, acc_sh, semI, semG, semS, seed):
    cid, sid, wid = _ids()
    base = sid * SLC
    if seed:
        @pl.when(cid == 0)
        def _():
            pltpu.sync_copy(table_hbm.at[pl.ds(base, SLC), :],
                            acc_sh.at[pl.ds(base, SLC), :])

        @pl.when(cid != 0)
        def _():
            _zero_acc(zb2, acc_sh, sid, semI)
    else:
        _zero_acc(zb2, acc_sh, sid, semI)
    plsc.subcore_barrier()

    def _idx(p, j):
        off = wid * (CH * CK) + j * CK
        ds = [
            pltpu.make_async_copy(ei_hbm.at[pl.ds(off, CK)], row8.at[p], semI),
            pltpu.make_async_copy(ei_hbm.at[pl.ds(E + off, CK)], col8.at[p], semI),
        ]
        if w8 is not None:
            ds.append(pltpu.make_async_copy(w3_hbm.at[wid, j], w8.at[p], semI))
        return ds

    def _gather(j, sg):
        return pltpu.make_async_copy(
            table_hbm.at[row8.at[jnp.bitwise_and(j, 7)]],
            buf4.at[jnp.bitwise_and(j, 3)], sg)

    def _scat(j, ss):
        return pltpu.make_async_copy(
            buf4.at[jnp.bitwise_and(j, 3)],
            acc_sh.at[col8.at[jnp.bitwise_and(j, 7)]], ss)

    def _scale(j):
        b = jnp.bitwise_and(j, 3)
        s = jnp.bitwise_and(j, 7)
        @plsc.parallel_loop(0, CK // 16)
        def _grp(k):
            wvec = w8[s, pl.ds(16 * k, 16)]
            for l in range(16):
                sc = wvec[l]
                bufe = buf4.at[b, 16 * k + l]
                vals = [bufe[pl.ds(16 * q, 16)] * sc for q in range(D // 16)]
                for q in range(D // 16):
                    bufe[pl.ds(16 * q, 16)] = vals[q]

    for j in range(6):
        for d in _idx(j, j):
            d.start()
    for d in _idx(0, 0):
        d.wait()
    for d in _idx(1, 1):
        d.wait()
    _gather(0, semG.at[0]).start()
    _gather(1, semG.at[1]).start()

    def _body(j, _):
        sg = jnp.bitwise_and(j, 1)

        @pl.when(j >= 2)
        def _():
            _scat(j - 2, semS.at[sg]).wait()

        @pl.when(j + 6 < CH)
        def _():
            for d in _idx(jnp.bitwise_and(j + 6, 7), j + 6):
                d.start()

        _gather(j, semG.at[sg]).wait()
        if w8 is not None:
            _scale(j)
        _scat(j, semS.at[sg]).start(add=True)

        @pl.when(j + 2 < CH)
        def _():
            for d in _idx(jnp.bitwise_and(j + 2, 7), j + 2):
                d.wait()
            _gather(j + 2, semG.at[sg]).start()
        return _

    lax.fori_loop(0, CH, _body, None)
    _scat(CH - 2, semS.at[1]).wait()
    _scat(CH - 1, semS.at[0]).wait()
    plsc.subcore_barrier()
    pltpu.sync_copy(acc_sh.at[pl.ds(base, SLC), :], out_ref.at[cid, pl.ds(base, SLC), :])


@functools.partial(
    pl.kernel,
    out_type=[jax.ShapeDtypeStruct((NC, N_ACC, D), _f32)],
    mesh=_mesh,
    scratch_types=[
        pltpu.VMEM((8, CK), jnp.int32),
        pltpu.VMEM((8, CK), jnp.int32),
        pltpu.VMEM((8, CK), _f32),
        pltpu.VMEM((4, CK, D), _f32),
        pltpu.VMEM((20, D), _f32),
        pltpu.VMEM_SHARED((N_ACC, D), _f32),
        pltpu.SemaphoreType.DMA,
        pltpu.SemaphoreType.DMA((2,)),
        pltpu.SemaphoreType.DMA((2,)),
    ],
)
def _pass_b(h2_hbm, ei_hbm, w3_hbm, acc_out,
            row8, col8, w8, buf4, zb2, acc_sh, semI, semG, semS):
    _agg_pass(h2_hbm, ei_hbm, w3_hbm, acc_out,
              row8, col8, w8, buf4, zb2, acc_sh, semI, semG, semS, True)


@functools.partial(
    pl.kernel,
    out_type=[jax.ShapeDtypeStruct((NC, N_ACC, D), _f32)],
    mesh=_mesh,
    scratch_types=[
        pltpu.VMEM((8, CK), jnp.int32),
        pltpu.VMEM((8, CK), jnp.int32),
        pltpu.VMEM((4, CK, D), _f32),
        pltpu.VMEM((20, D), _f32),
        pltpu.VMEM_SHARED((N_ACC, D), _f32),
        pltpu.SemaphoreType.DMA,
        pltpu.SemaphoreType.DMA((2,)),
        pltpu.SemaphoreType.DMA((2,)),
    ],
)
def _pass_c(gcn_hbm, ei_hbm, sum_out,
            row8, col8, buf4, zb2, acc_sh, semI, semG, semS):
    _agg_pass(gcn_hbm, ei_hbm, None, sum_out,
              row8, col8, None, buf4, zb2, acc_sh, semI, semG, semS, False)


_RB = 1000


def _tc12_body(x_ref, w_ref, degA_ref, degB_ref, h2_ref, dinv_ref):
    h = jnp.dot(x_ref[...], w_ref[...], preferred_element_type=_f32)
    deg = 1.0 + degA_ref[0] + degB_ref[0]
    dinv = lax.rsqrt(deg)
    h2_ref[...] = h * dinv
    dinv_ref[...] = dinv


_RB12 = 1024


def _tc12(x, W_gcn, degp3):
    return pl.pallas_call(
        _tc12_body,
        grid=(N_ACC // _RB12,),
        in_specs=[
            pl.BlockSpec((_RB12, D), lambda i: (i, 0)),
            pl.BlockSpec((D, D), lambda i: (0, 0)),
            pl.BlockSpec((1, _RB12, 1), lambda i: (0, i, 0)),
            pl.BlockSpec((1, _RB12, 1), lambda i: (1, i, 0)),
        ],
        out_specs=[
            pl.BlockSpec((_RB12, D), lambda i: (i, 0)),
            pl.BlockSpec((_RB12, 1), lambda i: (i, 0)),
        ],
        out_shape=[
            jax.ShapeDtypeStruct((N_ACC, D), _f32),
            jax.ShapeDtypeStruct((N_ACC, 1), _f32),
        ],
    )(x, W_gcn, degp3, degp3)


def _tc3_body(dinv_ref, accA_ref, accB_ref, bg_ref, gcn_ref):
    gcn_ref[...] = (dinv_ref[...] * (accA_ref[0] + accB_ref[0])
                    + bg_ref[...])


def _tc3(dinv, accp, bg):
    return pl.pallas_call(
        _tc3_body,
        grid=(N_T // _RB,),
        in_specs=[
            pl.BlockSpec((_RB, 1), lambda i: (i, 0)),
            pl.BlockSpec((1, _RB, D), lambda i: (0, i, 0)),
            pl.BlockSpec((1, _RB, D), lambda i: (1, i, 0)),
            pl.BlockSpec((1, D), lambda i: (0, 0)),
        ],
        out_specs=pl.BlockSpec((_RB, D), lambda i: (i, 0)),
        out_shape=jax.ShapeDtypeStruct((N_T, D), _f32),
    )(dinv, accp, accp, bg)


def _tc4_body(sA_ref, sB_ref, cntA_ref, cntB_ref, gcn_ref, wl_ref, wr_ref,
              bl_ref, out_ref):
    cnt = jnp.maximum(cntA_ref[0] + cntB_ref[0], 1.0)
    mean = (sA_ref[0] + sB_ref[0]) / cnt
    o = (lax.dot_general(mean, wl_ref[...], (((1,), (1,)), ((), ())),
                         preferred_element_type=_f32)
         + lax.dot_general(gcn_ref[...], wr_ref[...], (((1,), (1,)), ((), ())),
                           preferred_element_type=_f32)
         + bl_ref[...])
    ss = jnp.sum(o * o, axis=-1, keepdims=True)
    nrm = jnp.sqrt(jnp.maximum(ss, 1e-24))
    out_ref[...] = o / jnp.maximum(nrm, 1e-12)


def _tc4(sump, cntp3, gcn, Wl, Wr, bl):
    return pl.pallas_call(
        _tc4_body,
        grid=(N_T // _RB,),
        in_specs=[
            pl.BlockSpec((1, _RB, D), lambda i: (0, i, 0)),
            pl.BlockSpec((1, _RB, D), lambda i: (1, i, 0)),
            pl.BlockSpec((1, _RB, 1), lambda i: (0, i, 0)),
            pl.BlockSpec((1, _RB, 1), lambda i: (1, i, 0)),
            pl.BlockSpec((_RB, D), lambda i: (i, 0)),
            pl.BlockSpec((D, D), lambda i: (0, 0)),
            pl.BlockSpec((D, D), lambda i: (0, 0)),
            pl.BlockSpec((1, D), lambda i: (0, 0)),
        ],
        out_specs=pl.BlockSpec((_RB, D), lambda i: (i, 0)),
        out_shape=jax.ShapeDtypeStruct((N_T, D), _f32),
    )(sump, sump, cntp3, cntp3, gcn, Wl, Wr, bl)


def kernel(x, edge_index, e_id, attr, W_gcn, b_gcn, W_l, b_l, W_r):
    eidx = edge_index.reshape(2 * E)
    w3, degp, cntp = _pass_a(attr, e_id, eidx)
    h2, dinv = _tc12(x, W_gcn, degp[:, :, None])
    (accp,) = _pass_b(h2, eidx, w3)
    gcn = _tc3(dinv, accp, b_gcn[None, :])
    (sump,) = _pass_c(gcn, eidx)
    out = _tc4(sump, cntp[:, :, None], gcn, W_l, W_r, b_l[None, :])
    return out


# speedup vs baseline: 32.4418x; 1.0220x over previous
import functools

import jax
import jax.numpy as jnp
from jax import lax
from jax.experimental import pallas as pl
from jax.experimental.pallas import tpu as pltpu
from jax.experimental.pallas import tpu_sc as plsc

N_T = 10000
D = 128
E = 320000
NC = 2
NS = 16
NW = NC * NS
CK = 80
CH = 125
N_ACC = 10240
SLC = N_ACC // NS

_mesh = plsc.VectorSubcoreMesh(core_axis_name="c", subcore_axis_name="s")
_f32 = jnp.float32


def _ids():
    cid = lax.axis_index("c")
    sid = lax.axis_index("s")
    return cid, sid, sid * NC + cid


@functools.partial(
    pl.kernel,
    out_type=[
        jax.ShapeDtypeStruct((NW, CH, CK), _f32),
        jax.ShapeDtypeStruct((NC, N_ACC), _f32),
        jax.ShapeDtypeStruct((NC, N_ACC), _f32),
    ],
    mesh=_mesh,
    scratch_types=[
        pltpu.VMEM((8, CK), jnp.int32),
        pltpu.VMEM((8, CK), jnp.int32),
        pltpu.VMEM((CH, CK), _f32),
        pltpu.VMEM((CK,), _f32),
        pltpu.VMEM((SLC,), _f32),
        pltpu.VMEM_SHARED((N_ACC,), _f32),
        pltpu.VMEM_SHARED((N_ACC,), _f32),
        pltpu.SemaphoreType.DMA,
        pltpu.SemaphoreType.DMA((2,)),
        pltpu.SemaphoreType.DMA((2,)),
    ],
)
def _pass_a(attr_hbm, eid_hbm, ei_hbm, w3_out, degp_out, cntp_out,
            eid8, col8, w_v, ones_v, zb, deg_sh, cnt_sh,
            semI, semG, semS):
    cid, sid, wid = _ids()
    base = sid * SLC

    def _zb(i, _):
        zb[pl.ds(16 * i, 16)] = jnp.zeros((16,), _f32)
        return _
    lax.fori_loop(0, SLC // 16, _zb, None)
    for k in range(CK // 16):
        ones_v[pl.ds(16 * k, 16)] = jnp.ones((16,), _f32)
    pltpu.sync_copy(zb, deg_sh.at[pl.ds(base, SLC)])
    pltpu.sync_copy(zb, cnt_sh.at[pl.ds(base, SLC)])
    plsc.subcore_barrier()

    def _idx(p, j):
        off = wid * (CH * CK) + j * CK
        return [
            pltpu.make_async_copy(eid_hbm.at[pl.ds(off, CK)], eid8.at[p], semI),
            pltpu.make_async_copy(ei_hbm.at[pl.ds(E + off, CK)], col8.at[p], semI),
        ]

    def _gather(j, sg):
        return pltpu.make_async_copy(attr_hbm.at[eid8.at[jnp.bitwise_and(j, 7)]],
                                     w_v.at[j], sg)

    def _scats(j, ss):
        s = jnp.bitwise_and(j, 7)
        return [
            pltpu.make_async_copy(w_v.at[j], deg_sh.at[col8.at[s]], ss),
            pltpu.make_async_copy(ones_v, cnt_sh.at[col8.at[s]], ss),
        ]

    for j in range(6):
        for d in _idx(j, j):
            d.start()
    for d in _idx(0, 0):
        d.wait()
    for d in _idx(1, 1):
        d.wait()
    _gather(0, semG.at[0]).start()
    _gather(1, semG.at[1]).start()

    def _body(j, _):
        sg = jnp.bitwise_and(j, 1)

        @pl.when(j >= 2)
        def _():
            for d in _scats(j - 2, semS.at[sg]):
                d.wait()

        @pl.when(j + 6 < CH)
        def _():
            for d in _idx(jnp.bitwise_and(j + 6, 7), j + 6):
                d.start()

        _gather(j, semG.at[sg]).wait()
        for d in _scats(j, semS.at[sg]):
            d.start(add=True)

        @pl.when(j + 2 < CH)
        def _():
            for d in _idx(jnp.bitwise_and(j + 2, 7), j + 2):
                d.wait()
            _gather(j + 2, semG.at[sg]).start()
        return _

    lax.fori_loop(0, CH, _body, None)
    for d in _scats(CH - 2, semS.at[1]):
        d.wait()
    for d in _scats(CH - 1, semS.at[0]):
        d.wait()
    pltpu.sync_copy(w_v, w3_out.at[wid])
    plsc.subcore_barrier()
    pltpu.sync_copy(deg_sh.at[pl.ds(base, SLC)], degp_out.at[cid, pl.ds(base, SLC)])
    pltpu.sync_copy(cnt_sh.at[pl.ds(base, SLC)], cntp_out.at[cid, pl.ds(base, SLC)])


def _zero_acc(zb2, acc_sh, sid, sem):
    zr = zb2.shape[0]

    def _zb(i, _):
        for k in range(D // 16):
            zb2[i, pl.ds(16 * k, 16)] = jnp.zeros((16,), _f32)
        return _
    lax.fori_loop(0, zr, _zb, None)
    base = sid * SLC
    ds = []
    for t in range(SLC // zr):
        ds.append(pltpu.async_copy(zb2, acc_sh.at[pl.ds(base + t * zr, zr), :], sem))
    for d in ds:
        d.wait()


def _agg_pass(table_hbm, ei_hbm, w3_hbm, out_ref,
              row8, col8, w8, buf4, zb2, acc_sh, semI, semG, semS, seed):
    cid, sid, wid = _ids()
    base = sid * SLC
    if seed:
        @pl.when(cid == 0)
        def _():
            pltpu.sync_copy(table_hbm.at[pl.ds(base, SLC), :],
                            acc_sh.at[pl.ds(base, SLC), :])

        @pl.when(cid != 0)
        def _():
            _zero_acc(zb2, acc_sh, sid, semI)
    else:
        _zero_acc(zb2, acc_sh, sid, semI)
    plsc.subcore_barrier()

    def _idx(p, j):
        off = wid * (CH * CK) + j * CK
        ds = [
            pltpu.make_async_copy(ei_hbm.at[pl.ds(off, CK)], row8.at[p], semI),
            pltpu.make_async_copy(ei_hbm.at[pl.ds(E + off, CK)], col8.at[p], semI),
        ]
        if w8 is not None:
            ds.append(pltpu.make_async_copy(w3_hbm.at[wid, j], w8.at[p], semI))
        return ds

    def _gather(j, sg):
        return pltpu.make_async_copy(
            table_hbm.at[row8.at[jnp.bitwise_and(j, 7)]],
            buf4.at[jnp.bitwise_and(j, 3)], sg)

    def _scat(j, ss):
        return pltpu.make_async_copy(
            buf4.at[jnp.bitwise_and(j, 3)],
            acc_sh.at[col8.at[jnp.bitwise_and(j, 7)]], ss)

    def _scale(j):
        b = jnp.bitwise_and(j, 3)
        s = jnp.bitwise_and(j, 7)
        @plsc.parallel_loop(0, CK // 16)
        def _grp(k):
            wvec = w8[s, pl.ds(16 * k, 16)]
            for l in range(16):
                sc = wvec[l]
                bufe = buf4.at[b, 16 * k + l]
                vals = [bufe[pl.ds(16 * q, 16)] * sc for q in range(D // 16)]
                for q in range(D // 16):
                    bufe[pl.ds(16 * q, 16)] = vals[q]

    for j in range(6):
        for d in _idx(j, j):
            d.start()
    for d in _idx(0, 0):
        d.wait()
    for d in _idx(1, 1):
        d.wait()
    _gather(0, semG.at[0]).start()
    _gather(1, semG.at[1]).start()

    def _body(j, _):
        sg = jnp.bitwise_and(j, 1)

        @pl.when(j >= 2)
        def _():
            _scat(j - 2, semS.at[sg]).wait()

        @pl.when(j + 6 < CH)
        def _():
            for d in _idx(jnp.bitwise_and(j + 6, 7), j + 6):
                d.start()

        _gather(j, semG.at[sg]).wait()
        if w8 is not None:
            _scale(j)
        _scat(j, semS.at[sg]).start(add=True)

        @pl.when(j + 2 < CH)
        def _():
            for d in _idx(jnp.bitwise_and(j + 2, 7), j + 2):
                d.wait()
            _gather(j + 2, semG.at[sg]).start()
        return _

    lax.fori_loop(0, CH, _body, None)
    _scat(CH - 2, semS.at[1]).wait()
    _scat(CH - 1, semS.at[0]).wait()
    plsc.subcore_barrier()
    pltpu.sync_copy(acc_sh.at[pl.ds(base, SLC), :], out_ref.at[cid, pl.ds(base, SLC), :])


@functools.partial(
    pl.kernel,
    out_type=[jax.ShapeDtypeStruct((NC, N_ACC, D), _f32)],
    mesh=_mesh,
    scratch_types=[
        pltpu.VMEM((8, CK), jnp.int32),
        pltpu.VMEM((8, CK), jnp.int32),
        pltpu.VMEM((8, CK), _f32),
        pltpu.VMEM((4, CK, D), _f32),
        pltpu.VMEM((20, D), _f32),
        pltpu.VMEM_SHARED((N_ACC, D), _f32),
        pltpu.SemaphoreType.DMA,
        pltpu.SemaphoreType.DMA((2,)),
        pltpu.SemaphoreType.DMA((2,)),
    ],
)
def _pass_b(h2_hbm, ei_hbm, w3_hbm, acc_out,
            row8, col8, w8, buf4, zb2, acc_sh, semI, semG, semS):
    _agg_pass(h2_hbm, ei_hbm, w3_hbm, acc_out,
              row8, col8, w8, buf4, zb2, acc_sh, semI, semG, semS, True)


@functools.partial(
    pl.kernel,
    out_type=[jax.ShapeDtypeStruct((NC, N_ACC, D), _f32)],
    mesh=_mesh,
    scratch_types=[
        pltpu.VMEM((8, CK), jnp.int32),
        pltpu.VMEM((8, CK), jnp.int32),
        pltpu.VMEM((4, CK, D), _f32),
        pltpu.VMEM((20, D), _f32),
        pltpu.VMEM_SHARED((N_ACC, D), _f32),
        pltpu.SemaphoreType.DMA,
        pltpu.SemaphoreType.DMA((2,)),
        pltpu.SemaphoreType.DMA((2,)),
    ],
)
def _pass_c(gcn_hbm, ei_hbm, sum_out,
            row8, col8, buf4, zb2, acc_sh, semI, semG, semS):
    _agg_pass(gcn_hbm, ei_hbm, None, sum_out,
              row8, col8, None, buf4, zb2, acc_sh, semI, semG, semS, False)


_RB = 1000


def _tc12_body(x_ref, w_ref, degp_ref, h2_ref, dinv_ref):
    h = jnp.dot(x_ref[...], w_ref[...], preferred_element_type=_f32)
    deg = 1.0 + degp_ref[0:1] + degp_ref[1:2]
    dinv = jnp.transpose(lax.rsqrt(deg))
    h2_ref[...] = h * dinv
    dinv_ref[...] = dinv


_RB12 = 1024


def _tc12(x, W_gcn, degp3):
    return pl.pallas_call(
        _tc12_body,
        grid=(N_ACC // _RB12,),
        in_specs=[
            pl.BlockSpec((_RB12, D), lambda i: (i, 0)),
            pl.BlockSpec((D, D), lambda i: (0, 0)),
            pl.BlockSpec((2, _RB12), lambda i: (0, i)),
        ],
        out_specs=[
            pl.BlockSpec((_RB12, D), lambda i: (i, 0)),
            pl.BlockSpec((_RB12, 1), lambda i: (i, 0)),
        ],
        out_shape=[
            jax.ShapeDtypeStruct((N_ACC, D), _f32),
            jax.ShapeDtypeStruct((N_ACC, 1), _f32),
        ],
    )(x, W_gcn, degp3)


def _tc3_body(dinv_ref, accA_ref, accB_ref, bg_ref, gcn_ref):
    gcn_ref[...] = (dinv_ref[...] * (accA_ref[0] + accB_ref[0])
                    + bg_ref[...])


def _tc3(dinv, accp, bg):
    return pl.pallas_call(
        _tc3_body,
        grid=(N_T // _RB,),
        in_specs=[
            pl.BlockSpec((_RB, 1), lambda i: (i, 0)),
            pl.BlockSpec((1, _RB, D), lambda i: (0, i, 0)),
            pl.BlockSpec((1, _RB, D), lambda i: (1, i, 0)),
            pl.BlockSpec((1, D), lambda i: (0, 0)),
        ],
        out_specs=pl.BlockSpec((_RB, D), lambda i: (i, 0)),
        out_shape=jax.ShapeDtypeStruct((N_T, D), _f32),
    )(dinv, accp, accp, bg)


def _tc4_body(sA_ref, sB_ref, cntA_ref, cntB_ref, gcn_ref, wl_ref, wr_ref,
              bl_ref, out_ref):
    cnt = jnp.maximum(cntA_ref[0] + cntB_ref[0], 1.0)
    mean = (sA_ref[0] + sB_ref[0]) / cnt
    o = (lax.dot_general(mean, wl_ref[...], (((1,), (1,)), ((), ())),
                         preferred_element_type=_f32)
         + lax.dot_general(gcn_ref[...], wr_ref[...], (((1,), (1,)), ((), ())),
                           preferred_element_type=_f32)
         + bl_ref[...])
    ss = jnp.sum(o * o, axis=-1, keepdims=True)
    nrm = jnp.sqrt(jnp.maximum(ss, 1e-24))
    out_ref[...] = o / jnp.maximum(nrm, 1e-12)


def _tc4(sump, cntp3, gcn, Wl, Wr, bl):
    return pl.pallas_call(
        _tc4_body,
        grid=(N_T // _RB,),
        in_specs=[
            pl.BlockSpec((1, _RB, D), lambda i: (0, i, 0)),
            pl.BlockSpec((1, _RB, D), lambda i: (1, i, 0)),
            pl.BlockSpec((1, _RB, 1), lambda i: (0, i, 0)),
            pl.BlockSpec((1, _RB, 1), lambda i: (1, i, 0)),
            pl.BlockSpec((_RB, D), lambda i: (i, 0)),
            pl.BlockSpec((D, D), lambda i: (0, 0)),
            pl.BlockSpec((D, D), lambda i: (0, 0)),
            pl.BlockSpec((1, D), lambda i: (0, 0)),
        ],
        out_specs=pl.BlockSpec((_RB, D), lambda i: (i, 0)),
        out_shape=jax.ShapeDtypeStruct((N_T, D), _f32),
    )(sump, sump, cntp3, cntp3, gcn, Wl, Wr, bl)


def kernel(x, edge_index, e_id, attr, W_gcn, b_gcn, W_l, b_l, W_r):
    eidx = edge_index.reshape(2 * E)
    w3, degp, cntp = _pass_a(attr, e_id, eidx)
    h2, dinv = _tc12(x, W_gcn, degp)
    (accp,) = _pass_b(h2, eidx, w3)
    gcn = _tc3(dinv, accp, b_gcn[None, :])
    (sump,) = _pass_c(gcn, eidx)
    out = _tc4(sump, cntp[:, :, None], gcn, W_l, W_r, b_l[None, :])
    return out
